# Initial kernel scaffold; baseline (speedup 1.0000x reference)
import jax
import jax.numpy as jnp
from jax.experimental import pallas as pl


def kernel(pos, node, connections, output, mask, noise, mode, params):
    raise NotImplementedError("write your pallas kernel here")


# speedup vs baseline: 545.1006x; 545.1006x over previous
import functools

import jax
import jax.numpy as jnp
from jax import lax
from jax.experimental import pallas as pl
from jax.experimental.pallas import tpu as pltpu
from jax.experimental.pallas import tpu_sc as plsc

_NC = 2
_NS = 16
_NW = _NC * _NS
_H = 128


def _sc_gather(table, idx, ch):
    V, D = table.shape
    B = idx.shape[0]
    per_w = B // _NW
    n_ch = per_w // ch
    assert per_w % ch == 0 and B % _NW == 0
    mesh = plsc.VectorSubcoreMesh(core_axis_name="c", subcore_axis_name="s")

    @functools.partial(
        pl.kernel,
        mesh=mesh,
        out_type=jax.ShapeDtypeStruct((B, D), jnp.float32),
        scratch_types=[
            pltpu.VMEM((ch,), jnp.int32),
            pltpu.VMEM((ch, D), jnp.float32),
            pltpu.SemaphoreType.DMA,
        ],
    )
    def k(table_hbm, idx_hbm, out_hbm, ibuf, rbuf, sem):
        wid = lax.axis_index("s") * _NC + lax.axis_index("c")
        base = wid * per_w

        def chunk(j, carry):
            off = base + j * ch
            pltpu.sync_copy(idx_hbm.at[pl.ds(off, ch)], ibuf)
            pltpu.async_copy(table_hbm.at[ibuf], rbuf, sem).wait()
            pltpu.sync_copy(rbuf, out_hbm.at[pl.ds(off, ch)])
            return carry

        lax.fori_loop(0, n_ch, chunk, 0)

    return k(table, idx)


def _sc_scatter_add(rows, idx, n_seg, ch):
    B, D = rows.shape
    per_w = B // _NW
    n_ch = per_w // ch
    assert per_w % ch == 0 and n_seg % 8 == 0
    rpt = (n_seg // _NS) // 8 * 8
    last = n_seg - (_NS - 1) * rpt
    mesh = plsc.VectorSubcoreMesh(core_axis_name="c", subcore_axis_name="s")

    @functools.partial(
        pl.kernel,
        mesh=mesh,
        out_type=jax.ShapeDtypeStruct((_NC, n_seg, D), jnp.float32),
        scratch_types=[
            pltpu.VMEM((ch,), jnp.int32),
            pltpu.VMEM((ch, D), jnp.float32),
            pltpu.VMEM_SHARED((n_seg, D), jnp.float32),
            pltpu.SemaphoreType.DMA,
        ],
    )
    def k(rows_hbm, idx_hbm, zeros_hbm, out_hbm, ibuf, rbuf, acc, sem):
        cid = lax.axis_index("c")
        sid = lax.axis_index("s")
        wid = sid * _NC + cid
        my0 = sid * rpt

        @pl.when(sid < _NS - 1)
        def _():
            pltpu.sync_copy(zeros_hbm.at[pl.ds(0, rpt)],
                            acc.at[pl.ds(my0, rpt)])

        @pl.when(sid == _NS - 1)
        def _():
            pltpu.sync_copy(zeros_hbm, acc.at[pl.ds((_NS - 1) * rpt, last)])

        plsc.subcore_barrier()
        base = wid * per_w

        def chunk(j, carry):
            off = base + j * ch
            pltpu.sync_copy(idx_hbm.at[pl.ds(off, ch)], ibuf)
            pltpu.sync_copy(rows_hbm.at[pl.ds(off, ch)], rbuf)
            pltpu.sync_copy(rbuf, acc.at[ibuf], add=True)
            return carry

        lax.fori_loop(0, n_ch, chunk, 0)
        plsc.subcore_barrier()

        @pl.when(sid < _NS - 1)
        def _():
            pltpu.sync_copy(acc.at[pl.ds(my0, rpt)],
                            out_hbm.at[cid, pl.ds(my0, rpt)])

        @pl.when(sid == _NS - 1)
        def _():
            pltpu.sync_copy(acc.at[pl.ds((_NS - 1) * rpt, last)],
                            out_hbm.at[cid, pl.ds((_NS - 1) * rpt, last)])

    zeros = jnp.zeros((last, D), jnp.float32)
    return k(rows, idx, zeros)


def _ln(y, g, b):
    mu = jnp.mean(y, axis=-1, keepdims=True)
    yc = y - mu
    var = jnp.mean(yc * yc, axis=-1, keepdims=True)
    return yc * lax.rsqrt(var + 1e-5) * g + b


def _dot(a, b):
    return jnp.dot(a, b, preferred_element_type=jnp.float32)


def _mlp2_ln_body(x_ref, w1_ref, b1_ref, w2_ref, b2_ref, g_ref, bb_ref, o_ref):
    h = jnp.maximum(_dot(x_ref[...], w1_ref[...]) + b1_ref[...], 0.0)
    y = _dot(h, w2_ref[...]) + b2_ref[...]
    o_ref[...] = _ln(y, g_ref[...], bb_ref[...])


def _edge_enc_body(ps_ref, pr_ref, w1_ref, b1_ref, w2_ref, b2_ref, g_ref,
                   bb_ref, o_ref):
    d = pr_ref[...] - ps_ref[...]
    dist = jnp.sqrt(jnp.sum(d * d, axis=-1, keepdims=True))
    lane = lax.broadcasted_iota(jnp.int32, d.shape, 1)
    x = d + jnp.where(lane == 3, dist, 0.0)
    h = jnp.maximum(_dot(x, w1_ref[...]) + b1_ref[...], 0.0)
    y = _dot(h, w2_ref[...]) + b2_ref[...]
    o_ref[...] = _ln(y, g_ref[...], bb_ref[...])


def _edge_mlp_body(with_res, *refs):
    if with_res:
        (sf_ref, rf_ref, el_ref, ws_ref, wr_ref, we_ref, b1_ref, w2_ref,
         b2_ref, g_ref, bb_ref, ne_ref, eln_ref) = refs
    else:
        (sf_ref, rf_ref, el_ref, ws_ref, wr_ref, we_ref, b1_ref, w2_ref,
         b2_ref, g_ref, bb_ref, ne_ref) = refs
    el = el_ref[...]
    h = (_dot(sf_ref[...], ws_ref[...]) + _dot(rf_ref[...], wr_ref[...])
         + _dot(el, we_ref[...]) + b1_ref[...])
    h = jnp.maximum(h, 0.0)
    y = _dot(h, w2_ref[...]) + b2_ref[...]
    ne = _ln(y, g_ref[...], bb_ref[...])
    ne_ref[...] = ne
    if with_res:
        eln_ref[...] = ne + el


def _node_mlp_body(x_ref, a0_ref, a1_ref, wa_ref, wb_ref, b1_ref, w2_ref,
                   b2_ref, g_ref, bb_ref, o_ref):
    x = x_ref[...]
    acc = a0_ref[0] + a1_ref[0]
    h = jnp.maximum(_dot(x, wa_ref[...]) + _dot(acc, wb_ref[...])
                    + b1_ref[...], 0.0)
    y = _dot(h, w2_ref[...]) + b2_ref[...]
    o_ref[...] = _ln(y, g_ref[...], bb_ref[...]) + x


def _dec_body(x_ref, w1_ref, b1_ref, w2_ref, b2_ref, o_ref):
    h = jnp.maximum(_dot(x_ref[...], w1_ref[...]) + b1_ref[...], 0.0)
    o_ref[...] = _dot(h, w2_ref[...]) + b2_ref[...]


def _full(shape):
    return pl.BlockSpec(shape, lambda i: tuple(0 for _ in shape))


def _rows(blk, d, off=0):
    return pl.BlockSpec((blk, d), lambda i, off=off: (i + off, 0))


def _mlp2_ln(x, w1, b1, w2, b2, g, bb, blk):
    r, din = x.shape
    return pl.pallas_call(
        _mlp2_ln_body,
        grid=(r // blk,),
        in_specs=[_rows(blk, din), _full(w1.shape), _full((1, _H)),
                  _full((_H, _H)), _full((1, _H)), _full((1, _H)),
                  _full((1, _H))],
        out_specs=_rows(blk, _H),
        out_shape=jax.ShapeDtypeStruct((r, _H), jnp.float32),
    )(x, w1, b1.reshape(1, _H), w2, b2.reshape(1, _H), g.reshape(1, _H),
      bb.reshape(1, _H))


def _edge_enc(g1, e, w1, b1, w2, b2, g, bb, blk):
    noff = e // blk

    def pos_spec(off):
        return pl.BlockSpec((blk, _H), lambda i, off=off: (i + off, 1))

    return pl.pallas_call(
        _edge_enc_body,
        grid=(e // blk,),
        in_specs=[pos_spec(0), pos_spec(noff), _full((_H, _H)),
                  _full((1, _H)), _full((_H, _H)), _full((1, _H)),
                  _full((1, _H)), _full((1, _H))],
        out_specs=_rows(blk, _H),
        out_shape=jax.ShapeDtypeStruct((e, _H), jnp.float32),
    )(g1, g1, w1, b1.reshape(1, _H), w2, b2.reshape(1, _H),
      g.reshape(1, _H), bb.reshape(1, _H))


def _edge_mlp(gath, el, ws, wr, we, b1, w2, b2, g, bb, blk, with_res):
    e = el.shape[0]
    noff = e // blk
    out_shape = [jax.ShapeDtypeStruct((e, _H), jnp.float32)]
    out_specs = [_rows(blk, _H)]
    if with_res:
        out_shape.append(jax.ShapeDtypeStruct((e, _H), jnp.float32))
        out_specs.append(_rows(blk, _H))
    res = pl.pallas_call(
        functools.partial(_edge_mlp_body, with_res),
        grid=(e // blk,),
        in_specs=[_rows(blk, _H), _rows(blk, _H, off=noff), _rows(blk, _H),
                  _full((_H, _H)), _full((_H, _H)), _full((_H, _H)),
                  _full((1, _H)), _full((_H, _H)), _full((1, _H)),
                  _full((1, _H)), _full((1, _H))],
        out_specs=out_specs,
        out_shape=out_shape,
    )(gath, gath, el, ws, wr, we, b1.reshape(1, _H), w2, b2.reshape(1, _H),
      g.reshape(1, _H), bb.reshape(1, _H))
    return res if with_res else res[0]


def _node_mlp(x, acc2, wa, wb, b1, w2, b2, g, bb, blk):
    r = x.shape[0]

    def a_spec(c):
        return pl.BlockSpec((1, blk, _H), lambda i, c=c: (c, i, 0))

    return pl.pallas_call(
        _node_mlp_body,
        grid=(r // blk,),
        in_specs=[_rows(blk, _H), a_spec(0), a_spec(1), _full((_H, _H)),
                  _full((_H, _H)), _full((1, _H)), _full((_H, _H)),
                  _full((1, _H)), _full((1, _H)), _full((1, _H))],
        out_specs=_rows(blk, _H),
        out_shape=jax.ShapeDtypeStruct((r, _H), jnp.float32),
    )(x, acc2, acc2, wa, wb, b1.reshape(1, _H), w2, b2.reshape(1, _H),
      g.reshape(1, _H), bb.reshape(1, _H))


def _dec(x, w1, b1, w2, b2, blk):
    r = x.shape[0]
    dout = w2.shape[1]
    return pl.pallas_call(
        _dec_body,
        grid=(r // blk,),
        in_specs=[_rows(blk, _H), _full((_H, _H)), _full((1, _H)),
                  _full((_H, dout)), _full((1, dout))],
        out_specs=_rows(blk, dout),
        out_shape=jax.ShapeDtypeStruct((r, dout), jnp.float32),
    )(x, w1, b1.reshape(1, _H), w2, b2.reshape(1, dout))


def kernel(pos, node, connections, output, mask, noise, mode, params):
    p = params
    pos0 = pos[0]
    node0 = node[0]
    conn = connections[0]
    n = pos0.shape[0]
    e = conn.shape[0]
    senders = conn[:, 0]
    receivers = conn[:, 1]
    idx2 = jnp.concatenate([senders, receivers])

    nm, ns = p["node_mean"], p["node_std"]
    enc_n = p["enc_node"]
    w1n = enc_n["W"][0] / ns[:, None]
    b1n = enc_n["b"][0] - (nm / ns) @ enc_n["W"][0]
    em, es = p["edge_mean"], p["edge_std"]
    enc_e = p["enc_edge"]
    w1e4 = enc_e["W"][0] / es[:, None]
    b1e = enc_e["b"][0] - (em / es) @ enc_e["W"][0]
    w1e128 = jnp.zeros((_H, _H), jnp.float32).at[:4, :].set(w1e4)

    eblk = 2000
    nblk = 2000

    x_node = jnp.concatenate([node0, pos0], axis=-1)
    feat = _mlp2_ln(x_node, w1n, b1n, enc_n["W"][1], enc_n["b"][1],
                    enc_n["ln_g"], enc_n["ln_b"], nblk)

    pos128 = jnp.pad(pos0, ((0, 0), (0, _H - 3)))
    table1 = jnp.concatenate([feat, pos128], axis=1)
    g1 = _sc_gather(table1, idx2, ch=80)
    edge_lat = _edge_enc(g1, e, w1e128, b1e, enc_e["W"][1], enc_e["b"][1],
                         enc_e["ln_g"], enc_e["ln_b"], eblk)

    n_steps = len(p["blocks"])
    for t, bp in enumerate(p["blocks"]):
        gath = g1 if t == 0 else _sc_gather(feat, idx2, ch=80)
        w1 = bp["edge"]["W"][0]
        eres = t < n_steps - 1
        r = _edge_mlp(gath, edge_lat, w1[:_H], w1[_H:2 * _H], w1[2 * _H:],
                      bp["edge"]["b"][0], bp["edge"]["W"][1],
                      bp["edge"]["b"][1], bp["edge"]["ln_g"],
                      bp["edge"]["ln_b"], eblk, with_res=eres)
        if eres:
            ne, edge_lat = r
        else:
            ne = r
        acc2 = _sc_scatter_add(ne, receivers, n, ch=40)
        wn = bp["node"]["W"][0]
        feat = _node_mlp(feat, acc2, wn[:_H], wn[_H:],
                         bp["node"]["b"][0], bp["node"]["W"][1],
                         bp["node"]["b"][1], bp["node"]["ln_g"],
                         bp["node"]["ln_b"], nblk)

    out = _dec(feat, p["dec"]["W"][0], p["dec"]["b"][0], p["dec"]["W"][1],
               p["dec"]["b"][1], nblk)
    return out[None]


# speedup vs baseline: 762.1272x; 1.3981x over previous
import functools

import jax
import jax.numpy as jnp
from jax import lax
from jax.experimental import pallas as pl
from jax.experimental.pallas import tpu as pltpu
from jax.experimental.pallas import tpu_sc as plsc

_NC = 2
_NS = 16
_NW = _NC * _NS
_H = 128


def _sc_gather(table, idx, ch):
    V, D = table.shape
    B = idx.shape[0]
    per_w = B // _NW
    n_ch = -(-per_w // ch)
    if n_ch % 2 == 0:
        n_ch += 1
    assert B % _NW == 0 and per_w % 8 == 0 and ch % 8 == 0 and per_w >= ch
    assert n_ch >= 3
    mesh = plsc.VectorSubcoreMesh(core_axis_name="c", subcore_axis_name="s")

    @functools.partial(
        pl.kernel,
        mesh=mesh,
        out_type=jax.ShapeDtypeStruct((B, D), jnp.float32),
        scratch_types=[
            pltpu.VMEM((ch,), jnp.int32),
            pltpu.VMEM((ch,), jnp.int32),
            pltpu.VMEM((ch, D), jnp.float32),
            pltpu.VMEM((ch, D), jnp.float32),
            pltpu.SemaphoreType.DMA,
            pltpu.SemaphoreType.DMA,
        ],
    )
    def k(table_hbm, idx_hbm, out_hbm, ib0, ib1, rb0, rb1, sm0, sm1):
        wid = lax.axis_index("s") * _NC + lax.axis_index("c")
        base = wid * per_w

        def offs(j):
            return base + jnp.minimum(j * ch, per_w - ch)

        def start(j, ib, rb, sm):
            pltpu.sync_copy(idx_hbm.at[pl.ds(offs(j), ch)], ib)
            return pltpu.async_copy(table_hbm.at[ib], rb, sm)

        def drain0():
            pltpu.make_async_copy(table_hbm.at[pl.ds(0, ch)], rb0, sm0).wait()

        start(0, ib0, rb0, sm0)

        def body(p, carry):
            j = 2 * p
            h1 = start(j + 1, ib1, rb1, sm1)
            drain0()
            pltpu.sync_copy(rb0, out_hbm.at[pl.ds(offs(j), ch)])
            start(j + 2, ib0, rb0, sm0)
            h1.wait()
            pltpu.sync_copy(rb1, out_hbm.at[pl.ds(offs(j + 1), ch)])
            return carry

        lax.fori_loop(0, (n_ch - 1) // 2, body, 0)
        drain0()
        pltpu.sync_copy(rb0, out_hbm.at[pl.ds(offs(n_ch - 1), ch)])

    return k(table, idx)


def _sc_scatter_add(rows, idx, n_seg, ch):
    B, D = rows.shape
    per_w = B // _NW
    n_ch = per_w // ch
    assert per_w % ch == 0 and n_seg % 8 == 0
    assert n_ch % 2 == 1 and n_ch >= 3
    rpt = (n_seg // _NS) // 8 * 8
    last = n_seg - (_NS - 1) * rpt
    mesh = plsc.VectorSubcoreMesh(core_axis_name="c", subcore_axis_name="s")

    @functools.partial(
        pl.kernel,
        mesh=mesh,
        out_type=jax.ShapeDtypeStruct((_NC, n_seg, D), jnp.float32),
        scratch_types=[
            pltpu.VMEM((ch,), jnp.int32),
            pltpu.VMEM((ch,), jnp.int32),
            pltpu.VMEM((ch, D), jnp.float32),
            pltpu.VMEM((ch, D), jnp.float32),
            pltpu.VMEM_SHARED((n_seg, D), jnp.float32),
            pltpu.SemaphoreType.DMA,
            pltpu.SemaphoreType.DMA,
        ],
    )
    def k(rows_hbm, idx_hbm, zeros_hbm, out_hbm, ib0, ib1, rb0, rb1, acc,
          sm0, sm1):
        cid = lax.axis_index("c")
        sid = lax.axis_index("s")
        wid = sid * _NC + cid
        my0 = sid * rpt

        @pl.when(sid < _NS - 1)
        def _():
            pltpu.sync_copy(zeros_hbm.at[pl.ds(0, rpt)],
                            acc.at[pl.ds(my0, rpt)])

        @pl.when(sid == _NS - 1)
        def _():
            pltpu.sync_copy(zeros_hbm, acc.at[pl.ds((_NS - 1) * rpt, last)])

        plsc.subcore_barrier()
        base = wid * per_w

        def start(j, ib, rb, sm):
            off = base + j * ch
            pltpu.sync_copy(idx_hbm.at[pl.ds(off, ch)], ib)
            return pltpu.async_copy(rows_hbm.at[pl.ds(off, ch)], rb, sm)

        def drain0():
            pltpu.make_async_copy(rows_hbm.at[pl.ds(0, ch)], rb0, sm0).wait()

        start(0, ib0, rb0, sm0)

        def body(p, carry):
            j = 2 * p
            h1 = start(j + 1, ib1, rb1, sm1)
            drain0()
            pltpu.sync_copy(rb0, acc.at[ib0], add=True)
            start(j + 2, ib0, rb0, sm0)
            h1.wait()
            pltpu.sync_copy(rb1, acc.at[ib1], add=True)
            return carry

        lax.fori_loop(0, (n_ch - 1) // 2, body, 0)
        drain0()
        pltpu.sync_copy(rb0, acc.at[ib0], add=True)
        plsc.subcore_barrier()

        @pl.when(sid < _NS - 1)
        def _():
            pltpu.sync_copy(acc.at[pl.ds(my0, rpt)],
                            out_hbm.at[cid, pl.ds(my0, rpt)])

        @pl.when(sid == _NS - 1)
        def _():
            pltpu.sync_copy(acc.at[pl.ds((_NS - 1) * rpt, last)],
                            out_hbm.at[cid, pl.ds((_NS - 1) * rpt, last)])

    zeros = jnp.zeros((last, D), jnp.float32)
    return k(rows, idx, zeros)


def _ln(y, g, b):
    mu = jnp.mean(y, axis=-1, keepdims=True)
    yc = y - mu
    var = jnp.mean(yc * yc, axis=-1, keepdims=True)
    return yc * lax.rsqrt(var + 1e-5) * g + b


def _dot(a, b):
    return jnp.dot(a, b, preferred_element_type=jnp.float32)


def _mlp2_ln_body(x_ref, w1_ref, b1_ref, w2_ref, b2_ref, g_ref, bb_ref, o_ref):
    h = jnp.maximum(_dot(x_ref[...], w1_ref[...]) + b1_ref[...], 0.0)
    y = _dot(h, w2_ref[...]) + b2_ref[...]
    o_ref[...] = _ln(y, g_ref[...], bb_ref[...])


def _edge_enc_body(ps_ref, pr_ref, w1_ref, b1_ref, w2_ref, b2_ref, g_ref,
                   bb_ref, o_ref):
    d = pr_ref[...] - ps_ref[...]
    dist = jnp.sqrt(jnp.sum(d * d, axis=-1, keepdims=True))
    lane = lax.broadcasted_iota(jnp.int32, d.shape, 1)
    x = d + jnp.where(lane == 3, dist, 0.0)
    h = jnp.maximum(_dot(x, w1_ref[...]) + b1_ref[...], 0.0)
    y = _dot(h, w2_ref[...]) + b2_ref[...]
    o_ref[...] = _ln(y, g_ref[...], bb_ref[...])


def _edge_mlp_body(with_res, *refs):
    if with_res:
        (sf_ref, rf_ref, el_ref, ws_ref, wr_ref, we_ref, b1_ref, w2_ref,
         b2_ref, g_ref, bb_ref, ne_ref, eln_ref) = refs
    else:
        (sf_ref, rf_ref, el_ref, ws_ref, wr_ref, we_ref, b1_ref, w2_ref,
         b2_ref, g_ref, bb_ref, ne_ref) = refs
    el = el_ref[...]
    h = (_dot(sf_ref[...], ws_ref[...]) + _dot(rf_ref[...], wr_ref[...])
         + _dot(el, we_ref[...]) + b1_ref[...])
    h = jnp.maximum(h, 0.0)
    y = _dot(h, w2_ref[...]) + b2_ref[...]
    ne = _ln(y, g_ref[...], bb_ref[...])
    ne_ref[...] = ne
    if with_res:
        eln_ref[...] = ne + el


def _node_mlp_body(x_ref, a0_ref, a1_ref, wa_ref, wb_ref, b1_ref, w2_ref,
                   b2_ref, g_ref, bb_ref, o_ref):
    x = x_ref[...]
    acc = a0_ref[0] + a1_ref[0]
    h = jnp.maximum(_dot(x, wa_ref[...]) + _dot(acc, wb_ref[...])
                    + b1_ref[...], 0.0)
    y = _dot(h, w2_ref[...]) + b2_ref[...]
    o_ref[...] = _ln(y, g_ref[...], bb_ref[...]) + x


def _dec_body(x_ref, w1_ref, b1_ref, w2_ref, b2_ref, o_ref):
    h = jnp.maximum(_dot(x_ref[...], w1_ref[...]) + b1_ref[...], 0.0)
    o_ref[...] = _dot(h, w2_ref[...]) + b2_ref[...]


def _full(shape):
    return pl.BlockSpec(shape, lambda i: tuple(0 for _ in shape))


def _rows(blk, d, off=0):
    return pl.BlockSpec((blk, d), lambda i, off=off: (i + off, 0))


def _mlp2_ln(x, w1, b1, w2, b2, g, bb, blk):
    r, din = x.shape
    return pl.pallas_call(
        _mlp2_ln_body,
        grid=(r // blk,),
        in_specs=[_rows(blk, din), _full(w1.shape), _full((1, _H)),
                  _full((_H, _H)), _full((1, _H)), _full((1, _H)),
                  _full((1, _H))],
        out_specs=_rows(blk, _H),
        out_shape=jax.ShapeDtypeStruct((r, _H), jnp.float32),
    )(x, w1, b1.reshape(1, _H), w2, b2.reshape(1, _H), g.reshape(1, _H),
      bb.reshape(1, _H))


def _edge_enc(g1, e, w1, b1, w2, b2, g, bb, blk):
    noff = e // blk

    def pos_spec(off):
        return pl.BlockSpec((blk, _H), lambda i, off=off: (i + off, 1))

    return pl.pallas_call(
        _edge_enc_body,
        grid=(e // blk,),
        in_specs=[pos_spec(0), pos_spec(noff), _full((_H, _H)),
                  _full((1, _H)), _full((_H, _H)), _full((1, _H)),
                  _full((1, _H)), _full((1, _H))],
        out_specs=_rows(blk, _H),
        out_shape=jax.ShapeDtypeStruct((e, _H), jnp.float32),
    )(g1, g1, w1, b1.reshape(1, _H), w2, b2.reshape(1, _H),
      g.reshape(1, _H), bb.reshape(1, _H))


def _edge_mlp(gath, el, ws, wr, we, b1, w2, b2, g, bb, blk, with_res):
    e = el.shape[0]
    noff = e // blk
    out_shape = [jax.ShapeDtypeStruct((e, _H), jnp.float32)]
    out_specs = [_rows(blk, _H)]
    if with_res:
        out_shape.append(jax.ShapeDtypeStruct((e, _H), jnp.float32))
        out_specs.append(_rows(blk, _H))
    res = pl.pallas_call(
        functools.partial(_edge_mlp_body, with_res),
        grid=(e // blk,),
        in_specs=[_rows(blk, _H), _rows(blk, _H, off=noff), _rows(blk, _H),
                  _full((_H, _H)), _full((_H, _H)), _full((_H, _H)),
                  _full((1, _H)), _full((_H, _H)), _full((1, _H)),
                  _full((1, _H)), _full((1, _H))],
        out_specs=out_specs,
        out_shape=out_shape,
    )(gath, gath, el, ws, wr, we, b1.reshape(1, _H), w2, b2.reshape(1, _H),
      g.reshape(1, _H), bb.reshape(1, _H))
    return res if with_res else res[0]


def _node_mlp(x, acc2, wa, wb, b1, w2, b2, g, bb, blk):
    r = x.shape[0]

    def a_spec(c):
        return pl.BlockSpec((1, blk, _H), lambda i, c=c: (c, i, 0))

    return pl.pallas_call(
        _node_mlp_body,
        grid=(r // blk,),
        in_specs=[_rows(blk, _H), a_spec(0), a_spec(1), _full((_H, _H)),
                  _full((_H, _H)), _full((1, _H)), _full((_H, _H)),
                  _full((1, _H)), _full((1, _H)), _full((1, _H))],
        out_specs=_rows(blk, _H),
        out_shape=jax.ShapeDtypeStruct((r, _H), jnp.float32),
    )(x, acc2, acc2, wa, wb, b1.reshape(1, _H), w2, b2.reshape(1, _H),
      g.reshape(1, _H), bb.reshape(1, _H))


def _dec(x, w1, b1, w2, b2, blk):
    r = x.shape[0]
    dout = w2.shape[1]
    return pl.pallas_call(
        _dec_body,
        grid=(r // blk,),
        in_specs=[_rows(blk, _H), _full((_H, _H)), _full((1, _H)),
                  _full((_H, dout)), _full((1, dout))],
        out_specs=_rows(blk, dout),
        out_shape=jax.ShapeDtypeStruct((r, dout), jnp.float32),
    )(x, w1, b1.reshape(1, _H), w2, b2.reshape(1, dout))


def kernel(pos, node, connections, output, mask, noise, mode, params):
    p = params
    pos0 = pos[0]
    node0 = node[0]
    conn = connections[0]
    n = pos0.shape[0]
    e = conn.shape[0]
    senders = conn[:, 0]
    receivers = conn[:, 1]
    idx2 = jnp.concatenate([senders, receivers])

    nm, ns = p["node_mean"], p["node_std"]
    enc_n = p["enc_node"]
    w1n = enc_n["W"][0] / ns[:, None]
    b1n = enc_n["b"][0] - (nm / ns) @ enc_n["W"][0]
    em, es = p["edge_mean"], p["edge_std"]
    enc_e = p["enc_edge"]
    w1e4 = enc_e["W"][0] / es[:, None]
    b1e = enc_e["b"][0] - (em / es) @ enc_e["W"][0]
    w1e128 = jnp.zeros((_H, _H), jnp.float32).at[:4, :].set(w1e4)

    eblk = 2000
    nblk = 2000

    x_node = jnp.concatenate([node0, pos0], axis=-1)
    feat = _mlp2_ln(x_node, w1n, b1n, enc_n["W"][1], enc_n["b"][1],
                    enc_n["ln_g"], enc_n["ln_b"], nblk)

    pos128 = jnp.pad(pos0, ((0, 0), (0, _H - 3)))
    table1 = jnp.concatenate([feat, pos128], axis=1)
    g1 = _sc_gather(table1, idx2, ch=128)
    edge_lat = _edge_enc(g1, e, w1e128, b1e, enc_e["W"][1], enc_e["b"][1],
                         enc_e["ln_g"], enc_e["ln_b"], eblk)

    n_steps = len(p["blocks"])
    for t, bp in enumerate(p["blocks"]):
        gath = g1 if t == 0 else _sc_gather(feat, idx2, ch=128)
        w1 = bp["edge"]["W"][0]
        eres = t < n_steps - 1
        r = _edge_mlp(gath, edge_lat, w1[:_H], w1[_H:2 * _H], w1[2 * _H:],
                      bp["edge"]["b"][0], bp["edge"]["W"][1],
                      bp["edge"]["b"][1], bp["edge"]["ln_g"],
                      bp["edge"]["ln_b"], eblk, with_res=eres)
        if eres:
            ne, edge_lat = r
        else:
            ne = r
        acc2 = _sc_scatter_add(ne, receivers, n, ch=40)
        wn = bp["node"]["W"][0]
        feat = _node_mlp(feat, acc2, wn[:_H], wn[_H:],
                         bp["node"]["b"][0], bp["node"]["W"][1],
                         bp["node"]["b"][1], bp["node"]["ln_g"],
                         bp["node"]["ln_b"], nblk)

    out = _dec(feat, p["dec"]["W"][0], p["dec"]["b"][0], p["dec"]["W"][1],
               p["dec"]["b"][1], nblk)
    return out[None]


# speedup vs baseline: 837.0638x; 1.0983x over previous
import functools

import jax
import jax.numpy as jnp
from jax import lax
from jax.experimental import pallas as pl
from jax.experimental.pallas import tpu as pltpu
from jax.experimental.pallas import tpu_sc as plsc

_NC = 2
_NS = 16
_NW = _NC * _NS
_H = 128


def _sc_gather(table, idx, ch):
    V, D = table.shape
    B = idx.shape[0]
    per_w = B // _NW
    n_ch = -(-per_w // ch)
    if n_ch % 2 == 0:
        n_ch += 1
    assert B % _NW == 0 and per_w % 8 == 0 and ch % 8 == 0 and per_w >= ch
    assert n_ch >= 3
    mesh = plsc.VectorSubcoreMesh(core_axis_name="c", subcore_axis_name="s")

    @functools.partial(
        pl.kernel,
        mesh=mesh,
        out_type=jax.ShapeDtypeStruct((B, D), jnp.float32),
        scratch_types=[
            pltpu.VMEM((ch,), jnp.int32),
            pltpu.VMEM((ch,), jnp.int32),
            pltpu.VMEM((ch, D), jnp.float32),
            pltpu.VMEM((ch, D), jnp.float32),
            pltpu.SemaphoreType.DMA,
            pltpu.SemaphoreType.DMA,
        ],
    )
    def k(table_hbm, idx_hbm, out_hbm, ib0, ib1, rb0, rb1, sm0, sm1):
        wid = lax.axis_index("s") * _NC + lax.axis_index("c")
        base = wid * per_w

        def offs(j):
            return base + jnp.minimum(j * ch, per_w - ch)

        def start(j, ib, rb, sm):
            pltpu.sync_copy(idx_hbm.at[pl.ds(offs(j), ch)], ib)
            return pltpu.async_copy(table_hbm.at[ib], rb, sm)

        def drain0():
            pltpu.make_async_copy(table_hbm.at[pl.ds(0, ch)], rb0, sm0).wait()

        start(0, ib0, rb0, sm0)

        def body(p, carry):
            j = 2 * p
            h1 = start(j + 1, ib1, rb1, sm1)
            drain0()
            pltpu.sync_copy(rb0, out_hbm.at[pl.ds(offs(j), ch)])
            start(j + 2, ib0, rb0, sm0)
            h1.wait()
            pltpu.sync_copy(rb1, out_hbm.at[pl.ds(offs(j + 1), ch)])
            return carry

        lax.fori_loop(0, (n_ch - 1) // 2, body, 0)
        drain0()
        pltpu.sync_copy(rb0, out_hbm.at[pl.ds(offs(n_ch - 1), ch)])

    return k(table, idx)


def _sc_scatter_add(rows, idx, n_seg, ch):
    B, D = rows.shape
    per_w = B // _NW
    n_ch = per_w // ch
    assert per_w % ch == 0 and n_seg % 8 == 0
    assert n_ch % 2 == 1 and n_ch >= 3
    rpt = (n_seg // _NS) // 8 * 8
    last = n_seg - (_NS - 1) * rpt
    mesh = plsc.VectorSubcoreMesh(core_axis_name="c", subcore_axis_name="s")

    @functools.partial(
        pl.kernel,
        mesh=mesh,
        out_type=jax.ShapeDtypeStruct((_NC, n_seg, D), jnp.float32),
        scratch_types=[
            pltpu.VMEM((ch,), jnp.int32),
            pltpu.VMEM((ch,), jnp.int32),
            pltpu.VMEM((ch, D), jnp.float32),
            pltpu.VMEM((ch, D), jnp.float32),
            pltpu.VMEM_SHARED((n_seg, D), jnp.float32),
            pltpu.SemaphoreType.DMA,
            pltpu.SemaphoreType.DMA,
        ],
    )
    def k(rows_hbm, idx_hbm, zeros_hbm, out_hbm, ib0, ib1, rb0, rb1, acc,
          sm0, sm1):
        cid = lax.axis_index("c")
        sid = lax.axis_index("s")
        wid = sid * _NC + cid
        my0 = sid * rpt

        @pl.when(sid < _NS - 1)
        def _():
            pltpu.sync_copy(zeros_hbm.at[pl.ds(0, rpt)],
                            acc.at[pl.ds(my0, rpt)])

        @pl.when(sid == _NS - 1)
        def _():
            pltpu.sync_copy(zeros_hbm, acc.at[pl.ds((_NS - 1) * rpt, last)])

        plsc.subcore_barrier()
        base = wid * per_w

        def start(j, ib, rb, sm):
            off = base + j * ch
            pltpu.sync_copy(idx_hbm.at[pl.ds(off, ch)], ib)
            return pltpu.async_copy(rows_hbm.at[pl.ds(off, ch)], rb, sm)

        def drain0():
            pltpu.make_async_copy(rows_hbm.at[pl.ds(0, ch)], rb0, sm0).wait()

        start(0, ib0, rb0, sm0)

        def body(p, carry):
            j = 2 * p
            h1 = start(j + 1, ib1, rb1, sm1)
            drain0()
            pltpu.sync_copy(rb0, acc.at[ib0], add=True)
            start(j + 2, ib0, rb0, sm0)
            h1.wait()
            pltpu.sync_copy(rb1, acc.at[ib1], add=True)
            return carry

        lax.fori_loop(0, (n_ch - 1) // 2, body, 0)
        drain0()
        pltpu.sync_copy(rb0, acc.at[ib0], add=True)
        plsc.subcore_barrier()

        @pl.when(sid < _NS - 1)
        def _():
            pltpu.sync_copy(acc.at[pl.ds(my0, rpt)],
                            out_hbm.at[cid, pl.ds(my0, rpt)])

        @pl.when(sid == _NS - 1)
        def _():
            pltpu.sync_copy(acc.at[pl.ds((_NS - 1) * rpt, last)],
                            out_hbm.at[cid, pl.ds((_NS - 1) * rpt, last)])

    zeros = jnp.zeros((last, D), jnp.float32)
    return k(rows, idx, zeros)


def _ln(y, g, b):
    mu = jnp.mean(y, axis=-1, keepdims=True)
    yc = y - mu
    var = jnp.mean(yc * yc, axis=-1, keepdims=True)
    return yc * lax.rsqrt(var + 1e-5) * g + b


def _dot(a, b):
    return jnp.dot(a, b, preferred_element_type=jnp.float32)


def _enc_node_body(x_ref, w1_ref, b1_ref, w2_ref, b2_ref, g_ref, bb_ref,
                   o_ref):
    x = x_ref[...]
    h = jnp.maximum(_dot(x, w1_ref[...]) + b1_ref[...], 0.0)
    y = _dot(h, w2_ref[...]) + b2_ref[...]
    lat = _ln(y, g_ref[...], bb_ref[...])
    pospad = jnp.concatenate([x[:, -3:], jnp.zeros_like(x[:, 3:])], axis=-1)
    o_ref[...] = jnp.concatenate([lat, pospad], axis=-1)


def _edge_step1_body(ps_ref, pr_ref, sf_ref, rf_ref, ew1_ref, eb1_ref,
                     ew2_ref, eb2_ref, eg_ref, ebb_ref, ws_ref, wr_ref,
                     we_ref, b1_ref, w2_ref, b2_ref, g_ref, bb_ref,
                     ne_ref, eln_ref):
    d = pr_ref[...] - ps_ref[...]
    dist = jnp.sqrt(jnp.sum(d * d, axis=-1, keepdims=True))
    lane = lax.broadcasted_iota(jnp.int32, d.shape, 1)
    x = d + jnp.where(lane == 3, dist, 0.0)
    eh = jnp.maximum(_dot(x, ew1_ref[...]) + eb1_ref[...], 0.0)
    el = _ln(_dot(eh, ew2_ref[...]) + eb2_ref[...], eg_ref[...], ebb_ref[...])
    h = (_dot(sf_ref[...], ws_ref[...]) + _dot(rf_ref[...], wr_ref[...])
         + _dot(el, we_ref[...]) + b1_ref[...])
    h = jnp.maximum(h, 0.0)
    y = _dot(h, w2_ref[...]) + b2_ref[...]
    ne = _ln(y, g_ref[...], bb_ref[...])
    ne_ref[...] = ne
    eln_ref[...] = ne + el


def _edge_mlp_body(with_res, *refs):
    if with_res:
        (sf_ref, rf_ref, el_ref, ws_ref, wr_ref, we_ref, b1_ref, w2_ref,
         b2_ref, g_ref, bb_ref, ne_ref, eln_ref) = refs
    else:
        (sf_ref, rf_ref, el_ref, ws_ref, wr_ref, we_ref, b1_ref, w2_ref,
         b2_ref, g_ref, bb_ref, ne_ref) = refs
    el = el_ref[...]
    h = (_dot(sf_ref[...], ws_ref[...]) + _dot(rf_ref[...], wr_ref[...])
         + _dot(el, we_ref[...]) + b1_ref[...])
    h = jnp.maximum(h, 0.0)
    y = _dot(h, w2_ref[...]) + b2_ref[...]
    ne = _ln(y, g_ref[...], bb_ref[...])
    ne_ref[...] = ne
    if with_res:
        eln_ref[...] = ne + el


def _node_mlp_body(x_ref, a0_ref, a1_ref, wa_ref, wb_ref, b1_ref, w2_ref,
                   b2_ref, g_ref, bb_ref, o_ref):
    x = x_ref[...]
    acc = a0_ref[0] + a1_ref[0]
    h = jnp.maximum(_dot(x, wa_ref[...]) + _dot(acc, wb_ref[...])
                    + b1_ref[...], 0.0)
    y = _dot(h, w2_ref[...]) + b2_ref[...]
    o_ref[...] = _ln(y, g_ref[...], bb_ref[...]) + x


def _dec_body(x_ref, w1_ref, b1_ref, w2_ref, b2_ref, o_ref):
    h = jnp.maximum(_dot(x_ref[...], w1_ref[...]) + b1_ref[...], 0.0)
    o_ref[...] = _dot(h, w2_ref[...]) + b2_ref[...]


def _full(shape):
    return pl.BlockSpec(shape, lambda i: tuple(0 for _ in shape))


def _rows(blk, d, off=0):
    return pl.BlockSpec((blk, d), lambda i, off=off: (i + off, 0))


def _enc_node(x, w1, b1, w2, b2, g, bb, blk):
    r, din = x.shape
    return pl.pallas_call(
        _enc_node_body,
        grid=(r // blk,),
        in_specs=[_rows(blk, din), _full(w1.shape), _full((1, _H)),
                  _full((_H, _H)), _full((1, _H)), _full((1, _H)),
                  _full((1, _H))],
        out_specs=_rows(blk, 2 * _H),
        out_shape=jax.ShapeDtypeStruct((r, 2 * _H), jnp.float32),
    )(x, w1, b1.reshape(1, _H), w2, b2.reshape(1, _H), g.reshape(1, _H),
      bb.reshape(1, _H))


def _edge_step1(g1, e, enc_p, mlp_p, blk):
    noff = e // blk

    def lat_spec(off):
        return pl.BlockSpec((blk, _H), lambda i, off=off: (i + off, 0))

    def pos_spec(off):
        return pl.BlockSpec((blk, _H), lambda i, off=off: (i + off, 1))

    ew1, eb1, ew2, eb2, eg, ebb = enc_p
    ws, wr, we, b1, w2, b2, g, bb = mlp_p
    out_shape = [jax.ShapeDtypeStruct((e, _H), jnp.float32)] * 2
    return pl.pallas_call(
        _edge_step1_body,
        grid=(e // blk,),
        in_specs=[pos_spec(0), pos_spec(noff), lat_spec(0), lat_spec(noff)]
        + [_full((_H, _H)), _full((1, _H))] * 2 + [_full((1, _H))] * 2
        + [_full((_H, _H))] * 3 + [_full((1, _H)), _full((_H, _H)),
                                   _full((1, _H)), _full((1, _H)),
                                   _full((1, _H))],
        out_specs=[_rows(blk, _H), _rows(blk, _H)],
        out_shape=out_shape,
    )(g1, g1, g1, g1, ew1, eb1.reshape(1, _H), ew2, eb2.reshape(1, _H),
      eg.reshape(1, _H), ebb.reshape(1, _H), ws, wr, we,
      b1.reshape(1, _H), w2, b2.reshape(1, _H), g.reshape(1, _H),
      bb.reshape(1, _H))


def _edge_mlp(gath, el, ws, wr, we, b1, w2, b2, g, bb, blk, with_res):
    e = el.shape[0]
    noff = e // blk
    out_shape = [jax.ShapeDtypeStruct((e, _H), jnp.float32)]
    out_specs = [_rows(blk, _H)]
    if with_res:
        out_shape.append(jax.ShapeDtypeStruct((e, _H), jnp.float32))
        out_specs.append(_rows(blk, _H))
    res = pl.pallas_call(
        functools.partial(_edge_mlp_body, with_res),
        grid=(e // blk,),
        in_specs=[_rows(blk, _H), _rows(blk, _H, off=noff), _rows(blk, _H),
                  _full((_H, _H)), _full((_H, _H)), _full((_H, _H)),
                  _full((1, _H)), _full((_H, _H)), _full((1, _H)),
                  _full((1, _H)), _full((1, _H))],
        out_specs=out_specs,
        out_shape=out_shape,
    )(gath, gath, el, ws, wr, we, b1.reshape(1, _H), w2, b2.reshape(1, _H),
      g.reshape(1, _H), bb.reshape(1, _H))
    return res if with_res else res[0]


def _node_mlp(x, acc2, wa, wb, b1, w2, b2, g, bb, blk):
    r = x.shape[0]

    def a_spec(c):
        return pl.BlockSpec((1, blk, _H), lambda i, c=c: (c, i, 0))

    return pl.pallas_call(
        _node_mlp_body,
        grid=(r // blk,),
        in_specs=[_rows(blk, _H), a_spec(0), a_spec(1), _full((_H, _H)),
                  _full((_H, _H)), _full((1, _H)), _full((_H, _H)),
                  _full((1, _H)), _full((1, _H)), _full((1, _H))],
        out_specs=_rows(blk, _H),
        out_shape=jax.ShapeDtypeStruct((r, _H), jnp.float32),
    )(x, acc2, acc2, wa, wb, b1.reshape(1, _H), w2, b2.reshape(1, _H),
      g.reshape(1, _H), bb.reshape(1, _H))


def _dec(x, w1, b1, w2, b2, blk):
    r = x.shape[0]
    dout = w2.shape[1]
    return pl.pallas_call(
        _dec_body,
        grid=(r // blk,),
        in_specs=[_rows(blk, _H), _full((_H, _H)), _full((1, _H)),
                  _full((_H, dout)), _full((1, dout))],
        out_specs=_rows(blk, dout),
        out_shape=jax.ShapeDtypeStruct((r, dout), jnp.float32),
    )(x, w1, b1.reshape(1, _H), w2, b2.reshape(1, dout))


def kernel(pos, node, connections, output, mask, noise, mode, params):
    p = params
    pos0 = pos[0]
    node0 = node[0]
    conn = connections[0]
    n = pos0.shape[0]
    e = conn.shape[0]
    senders = conn[:, 0]
    receivers = conn[:, 1]
    idx2 = jnp.concatenate([senders, receivers])

    nm, ns = p["node_mean"], p["node_std"]
    enc_n = p["enc_node"]
    w1n = enc_n["W"][0] / ns[:, None]
    b1n = enc_n["b"][0] - (nm / ns) @ enc_n["W"][0]
    em, es = p["edge_mean"], p["edge_std"]
    enc_e = p["enc_edge"]
    w1e4 = enc_e["W"][0] / es[:, None]
    b1e = enc_e["b"][0] - (em / es) @ enc_e["W"][0]
    w1e128 = jnp.zeros((_H, _H), jnp.float32).at[:4, :].set(w1e4)

    eblk = 2000
    nblk = 2000

    x_node = jnp.concatenate([node0, pos0], axis=-1)
    feat = _enc_node(x_node, w1n, b1n, enc_n["W"][1], enc_n["b"][1],
                     enc_n["ln_g"], enc_n["ln_b"], nblk)

    n_steps = len(p["blocks"])
    edge_lat = None
    for t, bp in enumerate(p["blocks"]):
        gath = _sc_gather(feat, idx2, ch=128)
        w1 = bp["edge"]["W"][0]
        eres = t < n_steps - 1
        em_p = (w1[:_H], w1[_H:2 * _H], w1[2 * _H:], bp["edge"]["b"][0],
                bp["edge"]["W"][1], bp["edge"]["b"][1], bp["edge"]["ln_g"],
                bp["edge"]["ln_b"])
        if t == 0:
            enc_p = (w1e128, b1e, enc_e["W"][1], enc_e["b"][1],
                     enc_e["ln_g"], enc_e["ln_b"])
            ne, edge_lat = _edge_step1(gath, e, enc_p, em_p, eblk)
        else:
            r = _edge_mlp(gath, edge_lat, *em_p, eblk, with_res=eres)
            if eres:
                ne, edge_lat = r
            else:
                ne = r
        acc2 = _sc_scatter_add(ne, receivers, n, ch=40)
        wn = bp["node"]["W"][0]
        feat = _node_mlp(feat, acc2, wn[:_H], wn[_H:],
                         bp["node"]["b"][0], bp["node"]["W"][1],
                         bp["node"]["b"][1], bp["node"]["ln_g"],
                         bp["node"]["ln_b"], nblk)

    out = _dec(feat, p["dec"]["W"][0], p["dec"]["b"][0], p["dec"]["W"][1],
               p["dec"]["b"][1], nblk)
    return out[None]


# speedup vs baseline: 909.8513x; 1.0870x over previous
import functools

import jax
import jax.numpy as jnp
from jax import lax
from jax.experimental import pallas as pl
from jax.experimental.pallas import tpu as pltpu
from jax.experimental.pallas import tpu_sc as plsc

_NC = 2
_NS = 16
_NW = _NC * _NS
_H = 128


def _sc_gather(table, idx, ch):
    V, D = table.shape
    B = idx.shape[0]
    per_w = B // _NW
    n_ch = -(-per_w // ch)
    if n_ch % 2 == 0:
        n_ch += 1
    assert B % _NW == 0 and per_w % 8 == 0 and ch % 8 == 0 and per_w >= ch
    assert n_ch >= 3
    mesh = plsc.VectorSubcoreMesh(core_axis_name="c", subcore_axis_name="s")

    @functools.partial(
        pl.kernel,
        mesh=mesh,
        out_type=jax.ShapeDtypeStruct((B, D), jnp.float32),
        scratch_types=[
            pltpu.VMEM((ch,), jnp.int32),
            pltpu.VMEM((ch,), jnp.int32),
            pltpu.VMEM((ch, D), jnp.float32),
            pltpu.VMEM((ch, D), jnp.float32),
            pltpu.SemaphoreType.DMA,
            pltpu.SemaphoreType.DMA,
        ],
    )
    def k(table_hbm, idx_hbm, out_hbm, ib0, ib1, rb0, rb1, sm0, sm1):
        wid = lax.axis_index("s") * _NC + lax.axis_index("c")
        base = wid * per_w

        def offs(j):
            return base + jnp.minimum(j * ch, per_w - ch)

        def start(j, ib, rb, sm):
            pltpu.sync_copy(idx_hbm.at[pl.ds(offs(j), ch)], ib)
            return pltpu.async_copy(table_hbm.at[ib], rb, sm)

        def drain0():
            pltpu.make_async_copy(table_hbm.at[pl.ds(0, ch)], rb0, sm0).wait()

        start(0, ib0, rb0, sm0)

        def body(p, carry):
            j = 2 * p
            h1 = start(j + 1, ib1, rb1, sm1)
            drain0()
            pltpu.sync_copy(rb0, out_hbm.at[pl.ds(offs(j), ch)])
            start(j + 2, ib0, rb0, sm0)
            h1.wait()
            pltpu.sync_copy(rb1, out_hbm.at[pl.ds(offs(j + 1), ch)])
            return carry

        lax.fori_loop(0, (n_ch - 1) // 2, body, 0)
        drain0()
        pltpu.sync_copy(rb0, out_hbm.at[pl.ds(offs(n_ch - 1), ch)])

    return k(table, idx)


def _sc_scatter_add(rows, idx, n_seg, ch):
    B, D = rows.shape
    per_w = B // _NW
    n_ch = per_w // ch
    assert per_w % ch == 0 and n_seg % 8 == 0
    assert n_ch % 2 == 1 and n_ch >= 3
    n_acc = n_seg + 8
    rpt = (n_acc // _NS) // 8 * 8
    last = n_acc - (_NS - 1) * rpt
    mesh = plsc.VectorSubcoreMesh(core_axis_name="c", subcore_axis_name="s")

    @functools.partial(
        pl.kernel,
        mesh=mesh,
        out_type=jax.ShapeDtypeStruct((_NC, n_acc, D), jnp.float32),
        scratch_types=[
            pltpu.VMEM((ch,), jnp.int32),
            pltpu.VMEM((ch,), jnp.int32),
            pltpu.VMEM((ch, D), jnp.float32),
            pltpu.VMEM((ch, D), jnp.float32),
            pltpu.VMEM_SHARED((n_acc, D), jnp.float32),
            pltpu.SemaphoreType.DMA,
            pltpu.SemaphoreType.DMA,
        ],
    )
    def k(rows_hbm, idx_hbm, zeros_hbm, out_hbm, ib0, ib1, rb0, rb1, acc,
          sm0, sm1):
        cid = lax.axis_index("c")
        sid = lax.axis_index("s")
        wid = sid * _NC + cid
        my0 = sid * rpt

        @pl.when(sid < _NS - 1)
        def _():
            pltpu.sync_copy(zeros_hbm.at[pl.ds(0, rpt)],
                            acc.at[pl.ds(my0, rpt)])

        @pl.when(sid == _NS - 1)
        def _():
            pltpu.sync_copy(zeros_hbm, acc.at[pl.ds((_NS - 1) * rpt, last)])

        plsc.subcore_barrier()
        base = wid * per_w

        def start(j, ib, rb, sm):
            off = base + j * ch
            pltpu.sync_copy(idx_hbm.at[pl.ds(off, ch)], ib)
            return pltpu.async_copy(rows_hbm.at[pl.ds(off, ch)], rb, sm)

        def drain0():
            pltpu.make_async_copy(rows_hbm.at[pl.ds(0, ch)], rb0, sm0).wait()

        start(0, ib0, rb0, sm0)

        def body(p, carry):
            j = 2 * p
            h1 = start(j + 1, ib1, rb1, sm1)
            drain0()
            pltpu.sync_copy(rb0, acc.at[ib0], add=True)
            start(j + 2, ib0, rb0, sm0)
            h1.wait()
            pltpu.sync_copy(rb1, acc.at[ib1], add=True)
            return carry

        lax.fori_loop(0, (n_ch - 1) // 2, body, 0)
        drain0()
        pltpu.sync_copy(rb0, acc.at[ib0], add=True)
        plsc.subcore_barrier()

        @pl.when(sid < _NS - 1)
        def _():
            pltpu.sync_copy(acc.at[pl.ds(my0, rpt)],
                            out_hbm.at[cid, pl.ds(my0, rpt)])

        @pl.when(sid == _NS - 1)
        def _():
            pltpu.sync_copy(acc.at[pl.ds((_NS - 1) * rpt, last)],
                            out_hbm.at[cid, pl.ds((_NS - 1) * rpt, last)])

    zeros = jnp.zeros((last, D), jnp.float32)
    return k(rows, idx, zeros)


def _ln(y, g, b):
    mu = jnp.mean(y, axis=-1, keepdims=True)
    yc = y - mu
    var = jnp.mean(yc * yc, axis=-1, keepdims=True)
    return yc * lax.rsqrt(var + 1e-5) * g + b


def _dot(a, b):
    return jnp.dot(a, b, preferred_element_type=jnp.float32)


def _enc_node_body(x_ref, w1_ref, b1_ref, w2_ref, b2_ref, g_ref, bb_ref,
                   o_ref):
    x = x_ref[...]
    h = jnp.maximum(_dot(x, w1_ref[...]) + b1_ref[...], 0.0)
    y = _dot(h, w2_ref[...]) + b2_ref[...]
    lat = _ln(y, g_ref[...], bb_ref[...])
    pospad = jnp.concatenate([x[:, -3:], jnp.zeros_like(x[:, 3:])], axis=-1)
    o_ref[...] = jnp.concatenate([lat, pospad], axis=-1)


def _edge_step1_body(ps_ref, pr_ref, sf_ref, rf_ref, ew1_ref, eb1_ref,
                     ew2_ref, eb2_ref, eg_ref, ebb_ref, ws_ref, wr_ref,
                     we_ref, b1_ref, w2_ref, b2_ref, g_ref, bb_ref,
                     ne_ref, eln_ref):
    d = pr_ref[...] - ps_ref[...]
    dist = jnp.sqrt(jnp.sum(d * d, axis=-1, keepdims=True))
    lane = lax.broadcasted_iota(jnp.int32, d.shape, 1)
    x = d + jnp.where(lane == 3, dist, 0.0)
    eh = jnp.maximum(_dot(x, ew1_ref[...]) + eb1_ref[...], 0.0)
    el = _ln(_dot(eh, ew2_ref[...]) + eb2_ref[...], eg_ref[...], ebb_ref[...])
    h = (_dot(sf_ref[...], ws_ref[...]) + _dot(rf_ref[...], wr_ref[...])
         + _dot(el, we_ref[...]) + b1_ref[...])
    h = jnp.maximum(h, 0.0)
    y = _dot(h, w2_ref[...]) + b2_ref[...]
    ne = _ln(y, g_ref[...], bb_ref[...])
    ne_ref[...] = ne
    eln_ref[...] = ne + el


def _edge_mlp_body(with_res, *refs):
    if with_res:
        (sf_ref, rf_ref, el_ref, ws_ref, wr_ref, we_ref, b1_ref, w2_ref,
         b2_ref, g_ref, bb_ref, ne_ref, eln_ref) = refs
    else:
        (sf_ref, rf_ref, el_ref, ws_ref, wr_ref, we_ref, b1_ref, w2_ref,
         b2_ref, g_ref, bb_ref, ne_ref) = refs
    el = el_ref[...]
    h = (_dot(sf_ref[...], ws_ref[...]) + _dot(rf_ref[...], wr_ref[...])
         + _dot(el, we_ref[...]) + b1_ref[...])
    h = jnp.maximum(h, 0.0)
    y = _dot(h, w2_ref[...]) + b2_ref[...]
    ne = _ln(y, g_ref[...], bb_ref[...])
    ne_ref[...] = ne
    if with_res:
        eln_ref[...] = ne + el


def _node_mlp_body(x_ref, a0_ref, a1_ref, wa_ref, wb_ref, b1_ref, w2_ref,
                   b2_ref, g_ref, bb_ref, o_ref):
    x = x_ref[...]
    acc = a0_ref[0] + a1_ref[0]
    h = jnp.maximum(_dot(x, wa_ref[...]) + _dot(acc, wb_ref[...])
                    + b1_ref[...], 0.0)
    y = _dot(h, w2_ref[...]) + b2_ref[...]
    o_ref[...] = _ln(y, g_ref[...], bb_ref[...]) + x


def _dec_body(x_ref, w1_ref, b1_ref, w2_ref, b2_ref, o_ref):
    h = jnp.maximum(_dot(x_ref[...], w1_ref[...]) + b1_ref[...], 0.0)
    o_ref[...] = _dot(h, w2_ref[...]) + b2_ref[...]


def _full(shape):
    return pl.BlockSpec(shape, lambda i: tuple(0 for _ in shape))


def _rows(blk, d, off=0):
    return pl.BlockSpec((blk, d), lambda i, off=off: (i + off, 0))


def _enc_node(x, w1, b1, w2, b2, g, bb, blk):
    r, din = x.shape
    return pl.pallas_call(
        _enc_node_body,
        grid=(r // blk,),
        in_specs=[_rows(blk, din), _full(w1.shape), _full((1, _H)),
                  _full((_H, _H)), _full((1, _H)), _full((1, _H)),
                  _full((1, _H))],
        out_specs=_rows(blk, 2 * _H),
        out_shape=jax.ShapeDtypeStruct((r, 2 * _H), jnp.float32),
    )(x, w1, b1.reshape(1, _H), w2, b2.reshape(1, _H), g.reshape(1, _H),
      bb.reshape(1, _H))


def _edge_step1(g1, e, enc_p, mlp_p, blk, ne_rows):
    noff = e // blk

    def lat_spec(off):
        return pl.BlockSpec((blk, _H), lambda i, off=off: (i + off, 0))

    def pos_spec(off):
        return pl.BlockSpec((blk, _H), lambda i, off=off: (i + off, 1))

    ew1, eb1, ew2, eb2, eg, ebb = enc_p
    ws, wr, we, b1, w2, b2, g, bb = mlp_p
    out_shape = [jax.ShapeDtypeStruct((ne_rows, _H), jnp.float32),
                 jax.ShapeDtypeStruct((e, _H), jnp.float32)]
    return pl.pallas_call(
        _edge_step1_body,
        grid=(e // blk,),
        in_specs=[pos_spec(0), pos_spec(noff), lat_spec(0), lat_spec(noff)]
        + [_full((_H, _H)), _full((1, _H))] * 2 + [_full((1, _H))] * 2
        + [_full((_H, _H))] * 3 + [_full((1, _H)), _full((_H, _H)),
                                   _full((1, _H)), _full((1, _H)),
                                   _full((1, _H))],
        out_specs=[_rows(blk, _H), _rows(blk, _H)],
        out_shape=out_shape,
    )(g1, g1, g1, g1, ew1, eb1.reshape(1, _H), ew2, eb2.reshape(1, _H),
      eg.reshape(1, _H), ebb.reshape(1, _H), ws, wr, we,
      b1.reshape(1, _H), w2, b2.reshape(1, _H), g.reshape(1, _H),
      bb.reshape(1, _H))


def _edge_mlp(gath, el, ws, wr, we, b1, w2, b2, g, bb, blk, with_res,
              ne_rows):
    e = el.shape[0]
    noff = e // blk
    out_shape = [jax.ShapeDtypeStruct((ne_rows, _H), jnp.float32)]
    out_specs = [_rows(blk, _H)]
    if with_res:
        out_shape.append(jax.ShapeDtypeStruct((e, _H), jnp.float32))
        out_specs.append(_rows(blk, _H))
    res = pl.pallas_call(
        functools.partial(_edge_mlp_body, with_res),
        grid=(e // blk,),
        in_specs=[_rows(blk, _H), _rows(blk, _H, off=noff), _rows(blk, _H),
                  _full((_H, _H)), _full((_H, _H)), _full((_H, _H)),
                  _full((1, _H)), _full((_H, _H)), _full((1, _H)),
                  _full((1, _H)), _full((1, _H))],
        out_specs=out_specs,
        out_shape=out_shape,
    )(gath, gath, el, ws, wr, we, b1.reshape(1, _H), w2, b2.reshape(1, _H),
      g.reshape(1, _H), bb.reshape(1, _H))
    return res if with_res else res[0]


def _node_mlp(x, acc2, wa, wb, b1, w2, b2, g, bb, blk):
    r = x.shape[0]

    def a_spec(c):
        return pl.BlockSpec((1, blk, _H), lambda i, c=c: (c, i, 0))

    return pl.pallas_call(
        _node_mlp_body,
        grid=(r // blk,),
        in_specs=[_rows(blk, _H), a_spec(0), a_spec(1), _full((_H, _H)),
                  _full((_H, _H)), _full((1, _H)), _full((_H, _H)),
                  _full((1, _H)), _full((1, _H)), _full((1, _H))],
        out_specs=_rows(blk, _H),
        out_shape=jax.ShapeDtypeStruct((r, _H), jnp.float32),
    )(x, acc2, acc2, wa, wb, b1.reshape(1, _H), w2, b2.reshape(1, _H),
      g.reshape(1, _H), bb.reshape(1, _H))


def _dec(x, w1, b1, w2, b2, blk):
    r = x.shape[0]
    dout = w2.shape[1]
    return pl.pallas_call(
        _dec_body,
        grid=(r // blk,),
        in_specs=[_rows(blk, _H), _full((_H, _H)), _full((1, _H)),
                  _full((_H, dout)), _full((1, dout))],
        out_specs=_rows(blk, dout),
        out_shape=jax.ShapeDtypeStruct((r, dout), jnp.float32),
    )(x, w1, b1.reshape(1, _H), w2, b2.reshape(1, dout))


def kernel(pos, node, connections, output, mask, noise, mode, params):
    p = params
    pos0 = pos[0]
    node0 = node[0]
    conn = connections[0]
    n = pos0.shape[0]
    e = conn.shape[0]
    senders = conn[:, 0]
    receivers = conn[:, 1]
    idx2 = jnp.concatenate([senders, receivers])

    nm, ns = p["node_mean"], p["node_std"]
    enc_n = p["enc_node"]
    w1n = enc_n["W"][0] / ns[:, None]
    b1n = enc_n["b"][0] - (nm / ns) @ enc_n["W"][0]
    em, es = p["edge_mean"], p["edge_std"]
    enc_e = p["enc_edge"]
    w1e4 = enc_e["W"][0] / es[:, None]
    b1e = enc_e["b"][0] - (em / es) @ enc_e["W"][0]
    w1e128 = jnp.zeros((_H, _H), jnp.float32).at[:4, :].set(w1e4)

    eblk = 2000
    nblk = 2000

    x_node = jnp.concatenate([node0, pos0], axis=-1)
    feat = _enc_node(x_node, w1n, b1n, enc_n["W"][1], enc_n["b"][1],
                     enc_n["ln_g"], enc_n["ln_b"], nblk)

    n_steps = len(p["blocks"])
    edge_lat = None
    ch_s = 128
    n_ch_s = -(-(e // _NW) // ch_s) | 1
    b_pad = _NW * ch_s * n_ch_s
    recv_pad = jnp.pad(receivers, (0, b_pad - e), constant_values=n)
    for t, bp in enumerate(p["blocks"]):
        gath = _sc_gather(feat, idx2, ch=128)
        w1 = bp["edge"]["W"][0]
        eres = t < n_steps - 1
        em_p = (w1[:_H], w1[_H:2 * _H], w1[2 * _H:], bp["edge"]["b"][0],
                bp["edge"]["W"][1], bp["edge"]["b"][1], bp["edge"]["ln_g"],
                bp["edge"]["ln_b"])
        if t == 0:
            enc_p = (w1e128, b1e, enc_e["W"][1], enc_e["b"][1],
                     enc_e["ln_g"], enc_e["ln_b"])
            ne, edge_lat = _edge_step1(gath, e, enc_p, em_p, eblk, b_pad)
        else:
            r = _edge_mlp(gath, edge_lat, *em_p, eblk, with_res=eres,
                          ne_rows=b_pad)
            if eres:
                ne, edge_lat = r
            else:
                ne = r
        acc2 = _sc_scatter_add(ne, recv_pad, n, ch=ch_s)
        wn = bp["node"]["W"][0]
        feat = _node_mlp(feat, acc2, wn[:_H], wn[_H:],
                         bp["node"]["b"][0], bp["node"]["W"][1],
                         bp["node"]["b"][1], bp["node"]["ln_g"],
                         bp["node"]["ln_b"], nblk)

    out = _dec(feat, p["dec"]["W"][0], p["dec"]["b"][0], p["dec"]["W"][1],
               p["dec"]["b"][1], nblk)
    return out[None]


# speedup vs baseline: 1052.8270x; 1.1571x over previous
import functools

import jax
import jax.numpy as jnp
from jax import lax
from jax.experimental import pallas as pl
from jax.experimental.pallas import tpu as pltpu
from jax.experimental.pallas import tpu_sc as plsc

_NC = 2
_NS = 16
_NW = _NC * _NS
_H = 128


def _sc_gather(table, idx, ch):
    V, D = table.shape
    B = idx.shape[0]
    per_w = B // _NW
    n_ch = -(-per_w // ch)
    if n_ch % 2 == 0:
        n_ch += 1
    assert B % _NW == 0 and per_w % 8 == 0 and ch % 8 == 0 and per_w >= ch
    assert n_ch >= 3
    mesh = plsc.VectorSubcoreMesh(core_axis_name="c", subcore_axis_name="s")

    @functools.partial(
        pl.kernel,
        mesh=mesh,
        out_type=jax.ShapeDtypeStruct((B, D), table.dtype),
        scratch_types=[
            pltpu.VMEM((ch,), jnp.int32),
            pltpu.VMEM((ch,), jnp.int32),
            pltpu.VMEM((ch, D), table.dtype),
            pltpu.VMEM((ch, D), table.dtype),
            pltpu.SemaphoreType.DMA,
            pltpu.SemaphoreType.DMA,
        ],
    )
    def k(table_hbm, idx_hbm, out_hbm, ib0, ib1, rb0, rb1, sm0, sm1):
        wid = lax.axis_index("s") * _NC + lax.axis_index("c")
        base = wid * per_w

        def offs(j):
            return base + jnp.minimum(j * ch, per_w - ch)

        def start(j, ib, rb, sm):
            pltpu.sync_copy(idx_hbm.at[pl.ds(offs(j), ch)], ib)
            return pltpu.async_copy(table_hbm.at[ib], rb, sm)

        def drain0():
            pltpu.make_async_copy(table_hbm.at[pl.ds(0, ch)], rb0, sm0).wait()

        start(0, ib0, rb0, sm0)

        def body(p, carry):
            j = 2 * p
            h1 = start(j + 1, ib1, rb1, sm1)
            drain0()
            pltpu.sync_copy(rb0, out_hbm.at[pl.ds(offs(j), ch)])
            start(j + 2, ib0, rb0, sm0)
            h1.wait()
            pltpu.sync_copy(rb1, out_hbm.at[pl.ds(offs(j + 1), ch)])
            return carry

        lax.fori_loop(0, (n_ch - 1) // 2, body, 0)
        drain0()
        pltpu.sync_copy(rb0, out_hbm.at[pl.ds(offs(n_ch - 1), ch)])

    return k(table, idx)


def _sc_scatter_add(rows, idx, n_seg, ch):
    B, D = rows.shape
    per_w = B // _NW
    n_ch = per_w // ch
    assert per_w % ch == 0 and n_seg % 8 == 0
    assert n_ch % 2 == 1 and n_ch >= 3
    n_acc = n_seg + 8
    rpt = (n_acc // _NS) // 8 * 8
    last = n_acc - (_NS - 1) * rpt
    mesh = plsc.VectorSubcoreMesh(core_axis_name="c", subcore_axis_name="s")

    @functools.partial(
        pl.kernel,
        mesh=mesh,
        out_type=jax.ShapeDtypeStruct((_NC, n_acc, D), jnp.float32),
        scratch_types=[
            pltpu.VMEM((ch,), jnp.int32),
            pltpu.VMEM((ch,), jnp.int32),
            pltpu.VMEM((ch, D), jnp.float32),
            pltpu.VMEM((ch, D), jnp.float32),
            pltpu.VMEM_SHARED((n_acc, D), jnp.float32),
            pltpu.SemaphoreType.DMA,
            pltpu.SemaphoreType.DMA,
        ],
    )
    def k(rows_hbm, idx_hbm, zeros_hbm, out_hbm, ib0, ib1, rb0, rb1, acc,
          sm0, sm1):
        cid = lax.axis_index("c")
        sid = lax.axis_index("s")
        wid = sid * _NC + cid
        my0 = sid * rpt

        @pl.when(sid < _NS - 1)
        def _():
            pltpu.sync_copy(zeros_hbm.at[pl.ds(0, rpt)],
                            acc.at[pl.ds(my0, rpt)])

        @pl.when(sid == _NS - 1)
        def _():
            pltpu.sync_copy(zeros_hbm, acc.at[pl.ds((_NS - 1) * rpt, last)])

        plsc.subcore_barrier()
        base = wid * per_w

        def start(j, ib, rb, sm):
            off = base + j * ch
            pltpu.sync_copy(idx_hbm.at[pl.ds(off, ch)], ib)
            return pltpu.async_copy(rows_hbm.at[pl.ds(off, ch)], rb, sm)

        def drain0():
            pltpu.make_async_copy(rows_hbm.at[pl.ds(0, ch)], rb0, sm0).wait()

        start(0, ib0, rb0, sm0)

        def body(p, carry):
            j = 2 * p
            h1 = start(j + 1, ib1, rb1, sm1)
            drain0()
            pltpu.sync_copy(rb0, acc.at[ib0], add=True)
            start(j + 2, ib0, rb0, sm0)
            h1.wait()
            pltpu.sync_copy(rb1, acc.at[ib1], add=True)
            return carry

        lax.fori_loop(0, (n_ch - 1) // 2, body, 0)
        drain0()
        pltpu.sync_copy(rb0, acc.at[ib0], add=True)
        plsc.subcore_barrier()

        @pl.when(sid < _NS - 1)
        def _():
            pltpu.sync_copy(acc.at[pl.ds(my0, rpt)],
                            out_hbm.at[cid, pl.ds(my0, rpt)])

        @pl.when(sid == _NS - 1)
        def _():
            pltpu.sync_copy(acc.at[pl.ds((_NS - 1) * rpt, last)],
                            out_hbm.at[cid, pl.ds((_NS - 1) * rpt, last)])

    zeros = jnp.zeros((last, D), jnp.float32)
    return k(rows, idx, zeros)


def _ln(y, g, b):
    mu = jnp.mean(y, axis=-1, keepdims=True)
    yc = y - mu
    var = jnp.mean(yc * yc, axis=-1, keepdims=True)
    return yc * lax.rsqrt(var + 1e-5) * g + b


def _dot(a, b):
    return jnp.dot(a, b, preferred_element_type=jnp.float32)


def _pack_bf16_pair(lo, hi):
    def rnd(x):
        b = lax.bitcast_convert_type(x, jnp.int32)
        b = b + 0x7fff + (lax.shift_right_logical(b, 16) & 1)
        return lax.shift_right_logical(b, 16)
    return rnd(lo) | lax.shift_left(rnd(hi), 16)


def _unpack_bf16_pair(w):
    lo = lax.bitcast_convert_type(lax.shift_left(w, 16), jnp.float32)
    hi = lax.bitcast_convert_type(
        lax.shift_left(lax.shift_right_logical(w, 16), 16), jnp.float32)
    return lo, hi


def _enc_node_body(x_ref, w1_ref, b1_ref, w2_ref, b2_ref, g_ref, bb_ref,
                   o_ref, t_ref):
    x = x_ref[...]
    h = jnp.maximum(_dot(x, w1_ref[...]) + b1_ref[...], 0.0)
    y = _dot(h, w2_ref[...]) + b2_ref[...]
    lat = _ln(y, g_ref[...], bb_ref[...])
    pospad = jnp.concatenate([x[:, -3:], jnp.zeros_like(x[:, 3:])], axis=-1)
    o_ref[...] = lat
    t_ref[...] = _pack_bf16_pair(lat, pospad)


def _edge_step1_body(s_ref, r_ref, ew1_ref, eb1_ref,
                     ew2_ref, eb2_ref, eg_ref, ebb_ref, ws_ref, wr_ref,
                     we_ref, b1_ref, w2_ref, b2_ref, g_ref, bb_ref,
                     ne_ref, eln_ref):
    sf, sp = _unpack_bf16_pair(s_ref[...])
    rf, rp = _unpack_bf16_pair(r_ref[...])
    d = rp - sp
    dist = jnp.sqrt(jnp.sum(d * d, axis=-1, keepdims=True))
    lane = lax.broadcasted_iota(jnp.int32, d.shape, 1)
    x = d + jnp.where(lane == 3, dist, 0.0)
    eh = jnp.maximum(_dot(x, ew1_ref[...]) + eb1_ref[...], 0.0)
    el = _ln(_dot(eh, ew2_ref[...]) + eb2_ref[...], eg_ref[...], ebb_ref[...])
    h = (_dot(sf, ws_ref[...]) + _dot(rf, wr_ref[...])
         + _dot(el, we_ref[...]) + b1_ref[...])
    h = jnp.maximum(h, 0.0)
    y = _dot(h, w2_ref[...]) + b2_ref[...]
    ne = _ln(y, g_ref[...], bb_ref[...])
    ne_ref[...] = ne
    eln_ref[...] = ne + el


def _edge_mlp_body(with_res, *refs):
    if with_res:
        (sf_ref, rf_ref, el_ref, ws_ref, wr_ref, we_ref, b1_ref, w2_ref,
         b2_ref, g_ref, bb_ref, ne_ref, eln_ref) = refs
    else:
        (sf_ref, rf_ref, el_ref, ws_ref, wr_ref, we_ref, b1_ref, w2_ref,
         b2_ref, g_ref, bb_ref, ne_ref) = refs
    el = el_ref[...]
    h = (_dot(sf_ref[...], ws_ref[...]) + _dot(rf_ref[...], wr_ref[...])
         + _dot(el, we_ref[...]) + b1_ref[...])
    h = jnp.maximum(h, 0.0)
    y = _dot(h, w2_ref[...]) + b2_ref[...]
    ne = _ln(y, g_ref[...], bb_ref[...])
    ne_ref[...] = ne
    if with_res:
        eln_ref[...] = ne + el


def _node_feat(x_ref, a0_ref, a1_ref, wa_ref, wb_ref, b1_ref, w2_ref,
               b2_ref, g_ref, bb_ref):
    x = x_ref[...]
    acc = a0_ref[0] + a1_ref[0]
    h = jnp.maximum(_dot(x, wa_ref[...]) + _dot(acc, wb_ref[...])
                    + b1_ref[...], 0.0)
    y = _dot(h, w2_ref[...]) + b2_ref[...]
    return _ln(y, g_ref[...], bb_ref[...]) + x


def _node_mlp_body(x_ref, a0_ref, a1_ref, wa_ref, wb_ref, b1_ref, w2_ref,
                   b2_ref, g_ref, bb_ref, o_ref):
    o_ref[...] = _node_feat(x_ref, a0_ref, a1_ref, wa_ref, wb_ref, b1_ref,
                            w2_ref, b2_ref, g_ref, bb_ref)


def _node_dec_body(x_ref, a0_ref, a1_ref, wa_ref, wb_ref, b1_ref, w2_ref,
                   b2_ref, g_ref, bb_ref, dw1_ref, db1_ref, dw2_ref,
                   db2_ref, o_ref):
    f = _node_feat(x_ref, a0_ref, a1_ref, wa_ref, wb_ref, b1_ref, w2_ref,
                   b2_ref, g_ref, bb_ref)
    h = jnp.maximum(_dot(f, dw1_ref[...]) + db1_ref[...], 0.0)
    o_ref[...] = _dot(h, dw2_ref[...]) + db2_ref[...]


def _full(shape):
    return pl.BlockSpec(shape, lambda i: tuple(0 for _ in shape))


def _rows(blk, d, off=0):
    return pl.BlockSpec((blk, d), lambda i, off=off: (i + off, 0))


def _enc_node(x, w1, b1, w2, b2, g, bb, blk):
    r, din = x.shape
    return pl.pallas_call(
        _enc_node_body,
        grid=(r // blk,),
        in_specs=[_rows(blk, din), _full(w1.shape), _full((1, _H)),
                  _full((_H, _H)), _full((1, _H)), _full((1, _H)),
                  _full((1, _H))],
        out_specs=[_rows(blk, _H), _rows(blk, _H)],
        out_shape=[jax.ShapeDtypeStruct((r, _H), jnp.float32),
                   jax.ShapeDtypeStruct((r, _H), jnp.int32)],
    )(x, w1, b1.reshape(1, _H), w2, b2.reshape(1, _H), g.reshape(1, _H),
      bb.reshape(1, _H))


def _edge_step1(g1, e, enc_p, mlp_p, blk, ne_rows):
    noff = e // blk
    ew1, eb1, ew2, eb2, eg, ebb = enc_p
    ws, wr, we, b1, w2, b2, g, bb = mlp_p
    out_shape = [jax.ShapeDtypeStruct((ne_rows, _H), jnp.float32),
                 jax.ShapeDtypeStruct((e, _H), jnp.float32)]
    return pl.pallas_call(
        _edge_step1_body,
        grid=(e // blk,),
        in_specs=[_rows(blk, _H), _rows(blk, _H, off=noff)]
        + [_full((_H, _H)), _full((1, _H))] * 2 + [_full((1, _H))] * 2
        + [_full((_H, _H))] * 3 + [_full((1, _H)), _full((_H, _H)),
                                   _full((1, _H)), _full((1, _H)),
                                   _full((1, _H))],
        out_specs=[_rows(blk, _H), _rows(blk, _H)],
        out_shape=out_shape,
    )(g1, g1, ew1, eb1.reshape(1, _H), ew2, eb2.reshape(1, _H),
      eg.reshape(1, _H), ebb.reshape(1, _H), ws, wr, we,
      b1.reshape(1, _H), w2, b2.reshape(1, _H), g.reshape(1, _H),
      bb.reshape(1, _H))


def _edge_mlp(gath, el, ws, wr, we, b1, w2, b2, g, bb, blk, with_res,
              ne_rows):
    e = el.shape[0]
    noff = e // blk
    out_shape = [jax.ShapeDtypeStruct((ne_rows, _H), jnp.float32)]
    out_specs = [_rows(blk, _H)]
    if with_res:
        out_shape.append(jax.ShapeDtypeStruct((e, _H), jnp.float32))
        out_specs.append(_rows(blk, _H))
    res = pl.pallas_call(
        functools.partial(_edge_mlp_body, with_res),
        grid=(e // blk,),
        in_specs=[_rows(blk, _H), _rows(blk, _H, off=noff), _rows(blk, _H),
                  _full((_H, _H)), _full((_H, _H)), _full((_H, _H)),
                  _full((1, _H)), _full((_H, _H)), _full((1, _H)),
                  _full((1, _H)), _full((1, _H))],
        out_specs=out_specs,
        out_shape=out_shape,
    )(gath, gath, el, ws, wr, we, b1.reshape(1, _H), w2, b2.reshape(1, _H),
      g.reshape(1, _H), bb.reshape(1, _H))
    return res if with_res else res[0]


def _node_mlp(x, acc2, wa, wb, b1, w2, b2, g, bb, blk):
    r = x.shape[0]

    def a_spec(c):
        return pl.BlockSpec((1, blk, _H), lambda i, c=c: (c, i, 0))

    return pl.pallas_call(
        _node_mlp_body,
        grid=(r // blk,),
        in_specs=[_rows(blk, _H), a_spec(0), a_spec(1), _full((_H, _H)),
                  _full((_H, _H)), _full((1, _H)), _full((_H, _H)),
                  _full((1, _H)), _full((1, _H)), _full((1, _H))],
        out_specs=_rows(blk, _H),
        out_shape=jax.ShapeDtypeStruct((r, _H), jnp.float32),
    )(x, acc2, acc2, wa, wb, b1.reshape(1, _H), w2, b2.reshape(1, _H),
      g.reshape(1, _H), bb.reshape(1, _H))


def _node_dec(x, acc2, wa, wb, b1, w2, b2, g, bb, dw1, db1, dw2, db2, blk):
    r = x.shape[0]
    dout = dw2.shape[1]

    def a_spec(c):
        return pl.BlockSpec((1, blk, _H), lambda i, c=c: (c, i, 0))

    return pl.pallas_call(
        _node_dec_body,
        grid=(r // blk,),
        in_specs=[_rows(blk, _H), a_spec(0), a_spec(1), _full((_H, _H)),
                  _full((_H, _H)), _full((1, _H)), _full((_H, _H)),
                  _full((1, _H)), _full((1, _H)), _full((1, _H)),
                  _full((_H, _H)), _full((1, _H)), _full((_H, dout)),
                  _full((1, dout))],
        out_specs=_rows(blk, dout),
        out_shape=jax.ShapeDtypeStruct((r, dout), jnp.float32),
    )(x, acc2, acc2, wa, wb, b1.reshape(1, _H), w2, b2.reshape(1, _H),
      g.reshape(1, _H), bb.reshape(1, _H), dw1, db1.reshape(1, _H), dw2,
      db2.reshape(1, dout))


def kernel(pos, node, connections, output, mask, noise, mode, params):
    p = params
    pos0 = pos[0]
    node0 = node[0]
    conn = connections[0]
    n = pos0.shape[0]
    e = conn.shape[0]
    senders = conn[:, 0]
    receivers = conn[:, 1]
    idx2 = jnp.concatenate([senders, receivers])

    nm, ns = p["node_mean"], p["node_std"]
    enc_n = p["enc_node"]
    w1n = enc_n["W"][0] / ns[:, None]
    b1n = enc_n["b"][0] - (nm / ns) @ enc_n["W"][0]
    em, es = p["edge_mean"], p["edge_std"]
    enc_e = p["enc_edge"]
    w1e4 = enc_e["W"][0] / es[:, None]
    b1e = enc_e["b"][0] - (em / es) @ enc_e["W"][0]
    w1e128 = jnp.zeros((_H, _H), jnp.float32).at[:4, :].set(w1e4)

    eblk = 2000
    nblk = 2000

    x_node = jnp.concatenate([node0, pos0], axis=-1)
    feat, table = _enc_node(x_node, w1n, b1n, enc_n["W"][1], enc_n["b"][1],
                            enc_n["ln_g"], enc_n["ln_b"], nblk)

    n_steps = len(p["blocks"])
    edge_lat = None
    ch_s = 128
    n_ch_s = -(-(e // _NW) // ch_s) | 1
    b_pad = _NW * ch_s * n_ch_s
    recv_pad = jnp.pad(receivers, (0, b_pad - e), constant_values=n)
    out = None
    for t, bp in enumerate(p["blocks"]):
        gath = _sc_gather(table if t == 0 else feat, idx2, ch=128)
        w1 = bp["edge"]["W"][0]
        eres = t < n_steps - 1
        em_p = (w1[:_H], w1[_H:2 * _H], w1[2 * _H:], bp["edge"]["b"][0],
                bp["edge"]["W"][1], bp["edge"]["b"][1], bp["edge"]["ln_g"],
                bp["edge"]["ln_b"])
        if t == 0:
            enc_p = (w1e128, b1e, enc_e["W"][1], enc_e["b"][1],
                     enc_e["ln_g"], enc_e["ln_b"])
            ne, edge_lat = _edge_step1(gath, e, enc_p, em_p, eblk, b_pad)
        else:
            r = _edge_mlp(gath, edge_lat, *em_p, eblk, with_res=eres,
                          ne_rows=b_pad)
            if eres:
                ne, edge_lat = r
            else:
                ne = r
        acc2 = _sc_scatter_add(ne, recv_pad, n, ch=ch_s)
        wn = bp["node"]["W"][0]
        np_ = (wn[:_H], wn[_H:], bp["node"]["b"][0], bp["node"]["W"][1],
               bp["node"]["b"][1], bp["node"]["ln_g"], bp["node"]["ln_b"])
        if t < n_steps - 1:
            feat = _node_mlp(feat, acc2, *np_, nblk)
        else:
            out = _node_dec(feat, acc2, *np_, p["dec"]["W"][0],
                            p["dec"]["b"][0], p["dec"]["W"][1],
                            p["dec"]["b"][1], nblk)
    return out[None]


# speedup vs baseline: 1164.8775x; 1.1064x over previous
import functools

import jax
import jax.numpy as jnp
from jax import lax
from jax.experimental import pallas as pl
from jax.experimental.pallas import tpu as pltpu
from jax.experimental.pallas import tpu_sc as plsc

_NC = 2
_NS = 16
_NW = _NC * _NS
_H = 128


def _sc_gather(table, idx, ch):
    V, D = table.shape
    B = idx.shape[0]
    per_w = B // _NW
    n_ch = -(-per_w // ch)
    if n_ch % 2 == 0:
        n_ch += 1
    assert B % _NW == 0 and per_w % 8 == 0 and ch % 8 == 0 and per_w >= ch
    assert n_ch >= 3
    mesh = plsc.VectorSubcoreMesh(core_axis_name="c", subcore_axis_name="s")

    @functools.partial(
        pl.kernel,
        mesh=mesh,
        out_type=jax.ShapeDtypeStruct((B, D), table.dtype),
        scratch_types=[
            pltpu.VMEM((ch,), jnp.int32),
            pltpu.VMEM((ch,), jnp.int32),
            pltpu.VMEM((ch, D), table.dtype),
            pltpu.VMEM((ch, D), table.dtype),
            pltpu.SemaphoreType.DMA,
            pltpu.SemaphoreType.DMA,
        ],
    )
    def k(table_hbm, idx_hbm, out_hbm, ib0, ib1, rb0, rb1, sm0, sm1):
        wid = lax.axis_index("s") * _NC + lax.axis_index("c")
        base = wid * per_w

        def offs(j):
            return base + jnp.minimum(j * ch, per_w - ch)

        def start(j, ib, rb, sm):
            pltpu.sync_copy(idx_hbm.at[pl.ds(offs(j), ch)], ib)
            return pltpu.async_copy(table_hbm.at[ib], rb, sm)

        def drain0():
            pltpu.make_async_copy(table_hbm.at[pl.ds(0, ch)], rb0, sm0).wait()

        start(0, ib0, rb0, sm0)

        def body(p, carry):
            j = 2 * p
            h1 = start(j + 1, ib1, rb1, sm1)
            drain0()
            pltpu.sync_copy(rb0, out_hbm.at[pl.ds(offs(j), ch)])
            start(j + 2, ib0, rb0, sm0)
            h1.wait()
            pltpu.sync_copy(rb1, out_hbm.at[pl.ds(offs(j + 1), ch)])
            return carry

        lax.fori_loop(0, (n_ch - 1) // 2, body, 0)
        drain0()
        pltpu.sync_copy(rb0, out_hbm.at[pl.ds(offs(n_ch - 1), ch)])

    return k(table, idx)


def _sc_scatter_add(rows, idx, n_seg, ch):
    B, D = rows.shape
    per_w = B // _NW
    n_ch = per_w // ch
    assert per_w % ch == 0 and n_seg % 8 == 0
    assert n_ch % 2 == 1 and n_ch >= 3
    n_acc = n_seg + 8
    rpt = (n_acc // _NS) // 8 * 8
    last = n_acc - (_NS - 1) * rpt
    mesh = plsc.VectorSubcoreMesh(core_axis_name="c", subcore_axis_name="s")

    @functools.partial(
        pl.kernel,
        mesh=mesh,
        out_type=jax.ShapeDtypeStruct((_NC, n_acc, D), jnp.float32),
        scratch_types=[
            pltpu.VMEM((ch,), jnp.int32),
            pltpu.VMEM((ch,), jnp.int32),
            pltpu.VMEM((ch, D), jnp.float32),
            pltpu.VMEM((ch, D), jnp.float32),
            pltpu.VMEM_SHARED((n_acc, D), jnp.float32),
            pltpu.SemaphoreType.DMA,
            pltpu.SemaphoreType.DMA,
        ],
    )
    def k(rows_hbm, idx_hbm, zeros_hbm, out_hbm, ib0, ib1, rb0, rb1, acc,
          sm0, sm1):
        cid = lax.axis_index("c")
        sid = lax.axis_index("s")
        wid = sid * _NC + cid
        my0 = sid * rpt

        @pl.when(sid < _NS - 1)
        def _():
            pltpu.sync_copy(zeros_hbm.at[pl.ds(0, rpt)],
                            acc.at[pl.ds(my0, rpt)])

        @pl.when(sid == _NS - 1)
        def _():
            pltpu.sync_copy(zeros_hbm, acc.at[pl.ds((_NS - 1) * rpt, last)])

        plsc.subcore_barrier()
        base = wid * per_w

        def start(j, ib, rb, sm):
            off = base + j * ch
            pltpu.sync_copy(idx_hbm.at[pl.ds(off, ch)], ib)
            return pltpu.async_copy(rows_hbm.at[pl.ds(off, ch)], rb, sm)

        def drain0():
            pltpu.make_async_copy(rows_hbm.at[pl.ds(0, ch)], rb0, sm0).wait()

        start(0, ib0, rb0, sm0)

        def body(p, carry):
            j = 2 * p
            h1 = start(j + 1, ib1, rb1, sm1)
            drain0()
            pltpu.sync_copy(rb0, acc.at[ib0], add=True)
            start(j + 2, ib0, rb0, sm0)
            h1.wait()
            pltpu.sync_copy(rb1, acc.at[ib1], add=True)
            return carry

        lax.fori_loop(0, (n_ch - 1) // 2, body, 0)
        drain0()
        pltpu.sync_copy(rb0, acc.at[ib0], add=True)
        plsc.subcore_barrier()

        @pl.when(sid < _NS - 1)
        def _():
            pltpu.sync_copy(acc.at[pl.ds(my0, rpt)],
                            out_hbm.at[cid, pl.ds(my0, rpt)])

        @pl.when(sid == _NS - 1)
        def _():
            pltpu.sync_copy(acc.at[pl.ds((_NS - 1) * rpt, last)],
                            out_hbm.at[cid, pl.ds((_NS - 1) * rpt, last)])

    zeros = jnp.zeros((last, D), jnp.float32)
    return k(rows, idx, zeros)


def _ln(y, g, b):
    mu = jnp.mean(y, axis=-1, keepdims=True)
    yc = y - mu
    var = jnp.mean(yc * yc, axis=-1, keepdims=True)
    return yc * lax.rsqrt(var + 1e-5) * g + b


def _dot(a, b):
    return jnp.dot(a, b, preferred_element_type=jnp.float32)


def _pack_bf16_pair(lo, hi):
    def rnd(x):
        b = lax.bitcast_convert_type(x, jnp.int32)
        b = b + 0x7fff + (lax.shift_right_logical(b, 16) & 1)
        return lax.shift_right_logical(b, 16)
    return rnd(lo) | lax.shift_left(rnd(hi), 16)


def _unpack_bf16_pair(w):
    lo = lax.bitcast_convert_type(lax.shift_left(w, 16), jnp.float32)
    hi = lax.bitcast_convert_type(
        lax.shift_left(lax.shift_right_logical(w, 16), 16), jnp.float32)
    return lo, hi


def _enc_node_body(x_ref, w1_ref, b1_ref, w2_ref, b2_ref, g_ref, bb_ref,
                   o_ref, t_ref):
    x = x_ref[...]
    h = jnp.maximum(_dot(x, w1_ref[...]) + b1_ref[...], 0.0)
    y = _dot(h, w2_ref[...]) + b2_ref[...]
    lat = _ln(y, g_ref[...], bb_ref[...])
    pospad = jnp.concatenate([x[:, -3:], jnp.zeros_like(x[:, 3:])], axis=-1)
    o_ref[...] = lat
    t_ref[...] = _pack_bf16_pair(lat, pospad)


def _edge_step1_body(s_ref, r_ref, ew1_ref, eb1_ref,
                     ew2_ref, eb2_ref, eg_ref, ebb_ref, ws_ref, wr_ref,
                     we_ref, b1_ref, w2_ref, b2_ref, g_ref, bb_ref,
                     ne_ref, eln_ref):
    sf, sp = _unpack_bf16_pair(s_ref[...])
    rf, rp = _unpack_bf16_pair(r_ref[...])
    d = rp - sp
    dist = jnp.sqrt(jnp.sum(d * d, axis=-1, keepdims=True))
    lane = lax.broadcasted_iota(jnp.int32, d.shape, 1)
    x = d + jnp.where(lane == 3, dist, 0.0)
    eh = jnp.maximum(_dot(x, ew1_ref[...]) + eb1_ref[...], 0.0)
    el = _ln(_dot(eh, ew2_ref[...]) + eb2_ref[...], eg_ref[...], ebb_ref[...])
    h = (_dot(sf, ws_ref[...]) + _dot(rf, wr_ref[...])
         + _dot(el, we_ref[...]) + b1_ref[...])
    h = jnp.maximum(h, 0.0)
    y = _dot(h, w2_ref[...]) + b2_ref[...]
    ne = _ln(y, g_ref[...], bb_ref[...])
    ne_ref[...] = ne
    eln_ref[...] = ne + el


def _edge_mlp_body(with_res, *refs):
    if with_res:
        (sf_ref, rf_ref, el_ref, ws_ref, wr_ref, we_ref, b1_ref, w2_ref,
         b2_ref, g_ref, bb_ref, ne_ref, eln_ref) = refs
    else:
        (sf_ref, rf_ref, el_ref, ws_ref, wr_ref, we_ref, b1_ref, w2_ref,
         b2_ref, g_ref, bb_ref, ne_ref) = refs
    el = el_ref[...]
    h = (_dot(sf_ref[...], ws_ref[...]) + _dot(rf_ref[...], wr_ref[...])
         + _dot(el, we_ref[...]) + b1_ref[...])
    h = jnp.maximum(h, 0.0)
    y = _dot(h, w2_ref[...]) + b2_ref[...]
    ne = _ln(y, g_ref[...], bb_ref[...])
    ne_ref[...] = ne
    if with_res:
        eln_ref[...] = ne + el


def _node_feat(x_ref, a0_ref, a1_ref, a2_ref, a3_ref, wa_ref, wb_ref,
               b1_ref, w2_ref, b2_ref, g_ref, bb_ref):
    x = x_ref[...]
    acc = (a0_ref[0] + a1_ref[0]) + (a2_ref[0] + a3_ref[0])
    h = jnp.maximum(_dot(x, wa_ref[...]) + _dot(acc, wb_ref[...])
                    + b1_ref[...], 0.0)
    y = _dot(h, w2_ref[...]) + b2_ref[...]
    return _ln(y, g_ref[...], bb_ref[...]) + x


def _node_mlp_body(x_ref, a0_ref, a1_ref, a2_ref, a3_ref, wa_ref, wb_ref,
                   b1_ref, w2_ref, b2_ref, g_ref, bb_ref, o_ref):
    o_ref[...] = _node_feat(x_ref, a0_ref, a1_ref, a2_ref, a3_ref, wa_ref,
                            wb_ref, b1_ref, w2_ref, b2_ref, g_ref, bb_ref)


def _node_dec_body(x_ref, a0_ref, a1_ref, a2_ref, a3_ref, wa_ref, wb_ref,
                   b1_ref, w2_ref, b2_ref, g_ref, bb_ref, dw1_ref, db1_ref,
                   dw2_ref, db2_ref, o_ref):
    f = _node_feat(x_ref, a0_ref, a1_ref, a2_ref, a3_ref, wa_ref, wb_ref,
                   b1_ref, w2_ref, b2_ref, g_ref, bb_ref)
    h = jnp.maximum(_dot(f, dw1_ref[...]) + db1_ref[...], 0.0)
    o_ref[...] = _dot(h, dw2_ref[...]) + db2_ref[...]


def _full(shape):
    return pl.BlockSpec(shape, lambda i: tuple(0 for _ in shape))


def _rows(blk, d, off=0):
    return pl.BlockSpec((blk, d), lambda i, off=off: (i + off, 0))


def _enc_node(x, w1, b1, w2, b2, g, bb, blk):
    r, din = x.shape
    return pl.pallas_call(
        _enc_node_body,
        grid=(r // blk,),
        in_specs=[_rows(blk, din), _full(w1.shape), _full((1, _H)),
                  _full((_H, _H)), _full((1, _H)), _full((1, _H)),
                  _full((1, _H))],
        out_specs=[_rows(blk, _H), _rows(blk, _H)],
        out_shape=[jax.ShapeDtypeStruct((r, _H), jnp.float32),
                   jax.ShapeDtypeStruct((r, _H), jnp.int32)],
    )(x, w1, b1.reshape(1, _H), w2, b2.reshape(1, _H), g.reshape(1, _H),
      bb.reshape(1, _H))


def _edge_step1(g1, e, enc_p, mlp_p, blk, ne_rows):
    noff = e // blk
    ew1, eb1, ew2, eb2, eg, ebb = enc_p
    ws, wr, we, b1, w2, b2, g, bb = mlp_p
    out_shape = [jax.ShapeDtypeStruct((ne_rows, _H), jnp.float32),
                 jax.ShapeDtypeStruct((e, _H), jnp.float32)]
    return pl.pallas_call(
        _edge_step1_body,
        grid=(e // blk,),
        in_specs=[_rows(blk, _H), _rows(blk, _H, off=noff)]
        + [_full((_H, _H)), _full((1, _H))] * 2 + [_full((1, _H))] * 2
        + [_full((_H, _H))] * 3 + [_full((1, _H)), _full((_H, _H)),
                                   _full((1, _H)), _full((1, _H)),
                                   _full((1, _H))],
        out_specs=[_rows(blk, _H), _rows(blk, _H)],
        out_shape=out_shape,
    )(g1, g1, ew1, eb1.reshape(1, _H), ew2, eb2.reshape(1, _H),
      eg.reshape(1, _H), ebb.reshape(1, _H), ws, wr, we,
      b1.reshape(1, _H), w2, b2.reshape(1, _H), g.reshape(1, _H),
      bb.reshape(1, _H))


def _edge_mlp(gath, el, ws, wr, we, b1, w2, b2, g, bb, blk, with_res,
              ne_rows):
    e = el.shape[0]
    noff = e // blk
    out_shape = [jax.ShapeDtypeStruct((ne_rows, _H), jnp.float32)]
    out_specs = [_rows(blk, _H)]
    if with_res:
        out_shape.append(jax.ShapeDtypeStruct((e, _H), jnp.float32))
        out_specs.append(_rows(blk, _H))
    res = pl.pallas_call(
        functools.partial(_edge_mlp_body, with_res),
        grid=(e // blk,),
        in_specs=[_rows(blk, _H), _rows(blk, _H, off=noff), _rows(blk, _H),
                  _full((_H, _H)), _full((_H, _H)), _full((_H, _H)),
                  _full((1, _H)), _full((_H, _H)), _full((1, _H)),
                  _full((1, _H)), _full((1, _H))],
        out_specs=out_specs,
        out_shape=out_shape,
    )(gath, gath, el, ws, wr, we, b1.reshape(1, _H), w2, b2.reshape(1, _H),
      g.reshape(1, _H), bb.reshape(1, _H))
    return res if with_res else res[0]


def _a_spec(blk, c):
    return pl.BlockSpec((1, blk, _H), lambda i, c=c: (c, i, 0))


def _node_mlp(x, acc_a, acc_b, wa, wb, b1, w2, b2, g, bb, blk):
    r = x.shape[0]
    return pl.pallas_call(
        _node_mlp_body,
        grid=(r // blk,),
        in_specs=[_rows(blk, _H), _a_spec(blk, 0), _a_spec(blk, 1),
                  _a_spec(blk, 0), _a_spec(blk, 1), _full((_H, _H)),
                  _full((_H, _H)), _full((1, _H)), _full((_H, _H)),
                  _full((1, _H)), _full((1, _H)), _full((1, _H))],
        out_specs=_rows(blk, _H),
        out_shape=jax.ShapeDtypeStruct((r, _H), jnp.float32),
    )(x, acc_a, acc_a, acc_b, acc_b, wa, wb, b1.reshape(1, _H), w2,
      b2.reshape(1, _H), g.reshape(1, _H), bb.reshape(1, _H))


def _node_dec(x, acc_a, acc_b, wa, wb, b1, w2, b2, g, bb, dw1, db1, dw2,
              db2, blk):
    r = x.shape[0]
    dout = dw2.shape[1]
    return pl.pallas_call(
        _node_dec_body,
        grid=(r // blk,),
        in_specs=[_rows(blk, _H), _a_spec(blk, 0), _a_spec(blk, 1),
                  _a_spec(blk, 0), _a_spec(blk, 1), _full((_H, _H)),
                  _full((_H, _H)), _full((1, _H)), _full((_H, _H)),
                  _full((1, _H)), _full((1, _H)), _full((1, _H)),
                  _full((_H, _H)), _full((1, _H)), _full((_H, dout)),
                  _full((1, dout))],
        out_specs=_rows(blk, dout),
        out_shape=jax.ShapeDtypeStruct((r, dout), jnp.float32),
    )(x, acc_a, acc_a, acc_b, acc_b, wa, wb, b1.reshape(1, _H), w2,
      b2.reshape(1, _H), g.reshape(1, _H), bb.reshape(1, _H), dw1,
      db1.reshape(1, _H), dw2, db2.reshape(1, dout))


def kernel(pos, node, connections, output, mask, noise, mode, params):
    p = params
    pos0 = pos[0]
    node0 = node[0]
    conn = connections[0]
    n = pos0.shape[0]
    e = conn.shape[0]
    half = e // 2
    senders = conn[:, 0]
    receivers = conn[:, 1]
    idx2h = [jnp.concatenate([senders[:half], receivers[:half]]),
             jnp.concatenate([senders[half:], receivers[half:]])]

    nm, ns = p["node_mean"], p["node_std"]
    enc_n = p["enc_node"]
    w1n = enc_n["W"][0] / ns[:, None]
    b1n = enc_n["b"][0] - (nm / ns) @ enc_n["W"][0]
    em, es = p["edge_mean"], p["edge_std"]
    enc_e = p["enc_edge"]
    w1e4 = enc_e["W"][0] / es[:, None]
    b1e = enc_e["b"][0] - (em / es) @ enc_e["W"][0]
    w1e128 = jnp.zeros((_H, _H), jnp.float32).at[:4, :].set(w1e4)

    eblk = 2000
    nblk = 2000

    x_node = jnp.concatenate([node0, pos0], axis=-1)
    feat, table = _enc_node(x_node, w1n, b1n, enc_n["W"][1], enc_n["b"][1],
                            enc_n["ln_g"], enc_n["ln_b"], nblk)

    n_steps = len(p["blocks"])
    edge_lat = [None, None]
    ch_s = 128
    n_ch_s = -(-(half // _NW) // ch_s) | 1
    b_pad = _NW * ch_s * n_ch_s
    recv_pad = [jnp.pad(receivers[:half], (0, b_pad - half),
                        constant_values=n),
                jnp.pad(receivers[half:], (0, b_pad - half),
                        constant_values=n)]
    out = None
    enc_p = (w1e128, b1e, enc_e["W"][1], enc_e["b"][1],
             enc_e["ln_g"], enc_e["ln_b"])
    for t, bp in enumerate(p["blocks"]):
        tab = table if t == 0 else feat
        gath = [_sc_gather(tab, idx2h[0], ch=128),
                _sc_gather(tab, idx2h[1], ch=128)]
        w1 = bp["edge"]["W"][0]
        eres = t < n_steps - 1
        em_p = (w1[:_H], w1[_H:2 * _H], w1[2 * _H:], bp["edge"]["b"][0],
                bp["edge"]["W"][1], bp["edge"]["b"][1], bp["edge"]["ln_g"],
                bp["edge"]["ln_b"])
        nes = [None, None]
        for h in (0, 1):
            if t == 0:
                nes[h], edge_lat[h] = _edge_step1(gath[h], half, enc_p,
                                                  em_p, eblk, b_pad)
            else:
                r = _edge_mlp(gath[h], edge_lat[h], *em_p, eblk,
                              with_res=eres, ne_rows=b_pad)
                if eres:
                    nes[h], edge_lat[h] = r
                else:
                    nes[h] = r
        acc = [_sc_scatter_add(nes[0], recv_pad[0], n, ch=ch_s),
               _sc_scatter_add(nes[1], recv_pad[1], n, ch=ch_s)]
        wn = bp["node"]["W"][0]
        np_ = (wn[:_H], wn[_H:], bp["node"]["b"][0], bp["node"]["W"][1],
               bp["node"]["b"][1], bp["node"]["ln_g"], bp["node"]["ln_b"])
        if t < n_steps - 1:
            feat = _node_mlp(feat, acc[0], acc[1], *np_, nblk)
        else:
            out = _node_dec(feat, acc[0], acc[1], *np_, p["dec"]["W"][0],
                            p["dec"]["b"][0], p["dec"]["W"][1],
                            p["dec"]["b"][1], nblk)
    return out[None]


# speedup vs baseline: 1285.5129x; 1.1036x over previous
import functools

import jax
import jax.numpy as jnp
from jax import lax
from jax.experimental import pallas as pl
from jax.experimental.pallas import tpu as pltpu
from jax.experimental.pallas import tpu_sc as plsc

_NC = 2
_NS = 16
_NW = _NC * _NS
_H = 128


def _sc_gather(table, idx, ch):
    V, D = table.shape
    B = idx.shape[0]
    per_w = B // _NW
    n_ch = -(-per_w // ch)
    if n_ch % 2 == 0:
        n_ch += 1
    assert B % _NW == 0 and per_w % 8 == 0 and ch % 8 == 0 and per_w >= ch
    assert n_ch >= 3 and V % 8 == 0
    rpt = (V // _NS) // 8 * 8
    vlast = V - (_NS - 1) * rpt
    mesh = plsc.VectorSubcoreMesh(core_axis_name="c", subcore_axis_name="s")

    @functools.partial(
        pl.kernel,
        mesh=mesh,
        out_type=jax.ShapeDtypeStruct((B, D), table.dtype),
        scratch_types=[
            pltpu.VMEM((ch,), jnp.int32),
            pltpu.VMEM((ch,), jnp.int32),
            pltpu.VMEM((ch, D), table.dtype),
            pltpu.VMEM((ch, D), table.dtype),
            pltpu.VMEM_SHARED((V, D), table.dtype),
            pltpu.SemaphoreType.DMA,
            pltpu.SemaphoreType.DMA,
        ],
    )
    def k(table_hbm, idx_hbm, out_hbm, ib0, ib1, rb0, rb1, tab, sm0, sm1):
        sid = lax.axis_index("s")
        wid = sid * _NC + lax.axis_index("c")
        base = wid * per_w

        @pl.when(sid < _NS - 1)
        def _():
            pltpu.sync_copy(table_hbm.at[pl.ds(sid * rpt, rpt)],
                            tab.at[pl.ds(sid * rpt, rpt)])

        @pl.when(sid == _NS - 1)
        def _():
            pltpu.sync_copy(table_hbm.at[pl.ds((_NS - 1) * rpt, vlast)],
                            tab.at[pl.ds((_NS - 1) * rpt, vlast)])

        plsc.subcore_barrier()

        def offs(j):
            return base + jnp.minimum(j * ch, per_w - ch)

        def start(j, ib, rb, sm):
            pltpu.sync_copy(idx_hbm.at[pl.ds(offs(j), ch)], ib)
            return pltpu.async_copy(tab.at[ib], rb, sm)

        def drain0():
            pltpu.make_async_copy(table_hbm.at[pl.ds(0, ch)], rb0, sm0).wait()

        start(0, ib0, rb0, sm0)

        def body(p, carry):
            j = 2 * p
            h1 = start(j + 1, ib1, rb1, sm1)
            drain0()
            pltpu.sync_copy(rb0, out_hbm.at[pl.ds(offs(j), ch)])
            start(j + 2, ib0, rb0, sm0)
            h1.wait()
            pltpu.sync_copy(rb1, out_hbm.at[pl.ds(offs(j + 1), ch)])
            return carry

        lax.fori_loop(0, (n_ch - 1) // 2, body, 0)
        drain0()
        pltpu.sync_copy(rb0, out_hbm.at[pl.ds(offs(n_ch - 1), ch)])

    return k(table, idx)


def _sc_scatter_add(rows, idx, n_seg, ch):
    B, D = rows.shape
    per_w = B // _NW
    n_ch = per_w // ch
    assert per_w % ch == 0 and n_seg % 8 == 0
    assert n_ch % 2 == 1 and n_ch >= 3
    n_acc = n_seg + 8
    rpt = (n_acc // _NS) // 8 * 8
    last = n_acc - (_NS - 1) * rpt
    mesh = plsc.VectorSubcoreMesh(core_axis_name="c", subcore_axis_name="s")

    @functools.partial(
        pl.kernel,
        mesh=mesh,
        out_type=jax.ShapeDtypeStruct((_NC, n_acc, D), jnp.float32),
        scratch_types=[
            pltpu.VMEM((ch,), jnp.int32),
            pltpu.VMEM((ch,), jnp.int32),
            pltpu.VMEM((ch, D), jnp.float32),
            pltpu.VMEM((ch, D), jnp.float32),
            pltpu.VMEM_SHARED((n_acc, D), jnp.float32),
            pltpu.SemaphoreType.DMA,
            pltpu.SemaphoreType.DMA,
        ],
    )
    def k(rows_hbm, idx_hbm, zeros_hbm, out_hbm, ib0, ib1, rb0, rb1, acc,
          sm0, sm1):
        cid = lax.axis_index("c")
        sid = lax.axis_index("s")
        wid = sid * _NC + cid
        my0 = sid * rpt

        @pl.when(sid < _NS - 1)
        def _():
            pltpu.sync_copy(zeros_hbm.at[pl.ds(0, rpt)],
                            acc.at[pl.ds(my0, rpt)])

        @pl.when(sid == _NS - 1)
        def _():
            pltpu.sync_copy(zeros_hbm, acc.at[pl.ds((_NS - 1) * rpt, last)])

        plsc.subcore_barrier()
        base = wid * per_w

        def start(j, ib, rb, sm):
            off = base + j * ch
            pltpu.sync_copy(idx_hbm.at[pl.ds(off, ch)], ib)
            return pltpu.async_copy(rows_hbm.at[pl.ds(off, ch)], rb, sm)

        def drain0():
            pltpu.make_async_copy(rows_hbm.at[pl.ds(0, ch)], rb0, sm0).wait()

        start(0, ib0, rb0, sm0)

        def body(p, carry):
            j = 2 * p
            h1 = start(j + 1, ib1, rb1, sm1)
            drain0()
            pltpu.sync_copy(rb0, acc.at[ib0], add=True)
            start(j + 2, ib0, rb0, sm0)
            h1.wait()
            pltpu.sync_copy(rb1, acc.at[ib1], add=True)
            return carry

        lax.fori_loop(0, (n_ch - 1) // 2, body, 0)
        drain0()
        pltpu.sync_copy(rb0, acc.at[ib0], add=True)
        plsc.subcore_barrier()

        @pl.when(sid < _NS - 1)
        def _():
            pltpu.sync_copy(acc.at[pl.ds(my0, rpt)],
                            out_hbm.at[cid, pl.ds(my0, rpt)])

        @pl.when(sid == _NS - 1)
        def _():
            pltpu.sync_copy(acc.at[pl.ds((_NS - 1) * rpt, last)],
                            out_hbm.at[cid, pl.ds((_NS - 1) * rpt, last)])

    zeros = jnp.zeros((last, D), jnp.float32)
    return k(rows, idx, zeros)


def _ln(y, g, b):
    mu = jnp.mean(y, axis=-1, keepdims=True)
    yc = y - mu
    var = jnp.mean(yc * yc, axis=-1, keepdims=True)
    return yc * lax.rsqrt(var + 1e-5) * g + b


def _dot(a, b):
    return jnp.dot(a, b, preferred_element_type=jnp.float32)


def _pack_bf16_pair(lo, hi):
    def rnd(x):
        b = lax.bitcast_convert_type(x, jnp.int32)
        b = b + 0x7fff + (lax.shift_right_logical(b, 16) & 1)
        return lax.shift_right_logical(b, 16)
    return rnd(lo) | lax.shift_left(rnd(hi), 16)


def _unpack_bf16_pair(w):
    lo = lax.bitcast_convert_type(lax.shift_left(w, 16), jnp.float32)
    hi = lax.bitcast_convert_type(
        lax.shift_left(lax.shift_right_logical(w, 16), 16), jnp.float32)
    return lo, hi


def _enc_node_body(x_ref, w1_ref, b1_ref, w2_ref, b2_ref, g_ref, bb_ref,
                   o_ref, t_ref):
    x = x_ref[...]
    h = jnp.maximum(_dot(x, w1_ref[...]) + b1_ref[...], 0.0)
    y = _dot(h, w2_ref[...]) + b2_ref[...]
    lat = _ln(y, g_ref[...], bb_ref[...])
    pospad = jnp.concatenate([x[:, -3:], jnp.zeros_like(x[:, 3:])], axis=-1)
    o_ref[...] = lat
    t_ref[...] = _pack_bf16_pair(lat, pospad)


def _edge_step1_body(s_ref, r_ref, ew1_ref, eb1_ref,
                     ew2_ref, eb2_ref, eg_ref, ebb_ref, ws_ref, wr_ref,
                     we_ref, b1_ref, w2_ref, b2_ref, g_ref, bb_ref,
                     ne_ref, eln_ref):
    sf, sp = _unpack_bf16_pair(s_ref[...])
    rf, rp = _unpack_bf16_pair(r_ref[...])
    d = rp - sp
    dist = jnp.sqrt(jnp.sum(d * d, axis=-1, keepdims=True))
    lane = lax.broadcasted_iota(jnp.int32, d.shape, 1)
    x = d + jnp.where(lane == 3, dist, 0.0)
    eh = jnp.maximum(_dot(x, ew1_ref[...]) + eb1_ref[...], 0.0)
    el = _ln(_dot(eh, ew2_ref[...]) + eb2_ref[...], eg_ref[...], ebb_ref[...])
    h = (_dot(sf, ws_ref[...]) + _dot(rf, wr_ref[...])
         + _dot(el, we_ref[...]) + b1_ref[...])
    h = jnp.maximum(h, 0.0)
    y = _dot(h, w2_ref[...]) + b2_ref[...]
    ne = _ln(y, g_ref[...], bb_ref[...])
    ne_ref[...] = ne
    eln_ref[...] = ne + el


def _edge_mlp_body(with_res, *refs):
    if with_res:
        (sf_ref, rf_ref, el_ref, ws_ref, wr_ref, we_ref, b1_ref, w2_ref,
         b2_ref, g_ref, bb_ref, ne_ref, eln_ref) = refs
    else:
        (sf_ref, rf_ref, el_ref, ws_ref, wr_ref, we_ref, b1_ref, w2_ref,
         b2_ref, g_ref, bb_ref, ne_ref) = refs
    el = el_ref[...]
    h = (_dot(sf_ref[...], ws_ref[...]) + _dot(rf_ref[...], wr_ref[...])
         + _dot(el, we_ref[...]) + b1_ref[...])
    h = jnp.maximum(h, 0.0)
    y = _dot(h, w2_ref[...]) + b2_ref[...]
    ne = _ln(y, g_ref[...], bb_ref[...])
    ne_ref[...] = ne
    if with_res:
        eln_ref[...] = ne + el


def _node_feat(x_ref, a0_ref, a1_ref, a2_ref, a3_ref, wa_ref, wb_ref,
               b1_ref, w2_ref, b2_ref, g_ref, bb_ref):
    x = x_ref[...]
    acc = (a0_ref[0] + a1_ref[0]) + (a2_ref[0] + a3_ref[0])
    h = jnp.maximum(_dot(x, wa_ref[...]) + _dot(acc, wb_ref[...])
                    + b1_ref[...], 0.0)
    y = _dot(h, w2_ref[...]) + b2_ref[...]
    return _ln(y, g_ref[...], bb_ref[...]) + x


def _node_mlp_body(x_ref, a0_ref, a1_ref, a2_ref, a3_ref, wa_ref, wb_ref,
                   b1_ref, w2_ref, b2_ref, g_ref, bb_ref, o_ref):
    o_ref[...] = _node_feat(x_ref, a0_ref, a1_ref, a2_ref, a3_ref, wa_ref,
                            wb_ref, b1_ref, w2_ref, b2_ref, g_ref, bb_ref)


def _node_dec_body(x_ref, a0_ref, a1_ref, a2_ref, a3_ref, wa_ref, wb_ref,
                   b1_ref, w2_ref, b2_ref, g_ref, bb_ref, dw1_ref, db1_ref,
                   dw2_ref, db2_ref, o_ref):
    f = _node_feat(x_ref, a0_ref, a1_ref, a2_ref, a3_ref, wa_ref, wb_ref,
                   b1_ref, w2_ref, b2_ref, g_ref, bb_ref)
    h = jnp.maximum(_dot(f, dw1_ref[...]) + db1_ref[...], 0.0)
    o_ref[...] = _dot(h, dw2_ref[...]) + db2_ref[...]


def _full(shape):
    return pl.BlockSpec(shape, lambda i: tuple(0 for _ in shape))


def _rows(blk, d, off=0):
    return pl.BlockSpec((blk, d), lambda i, off=off: (i + off, 0))


def _enc_node(x, w1, b1, w2, b2, g, bb, blk):
    r, din = x.shape
    return pl.pallas_call(
        _enc_node_body,
        grid=(r // blk,),
        in_specs=[_rows(blk, din), _full(w1.shape), _full((1, _H)),
                  _full((_H, _H)), _full((1, _H)), _full((1, _H)),
                  _full((1, _H))],
        out_specs=[_rows(blk, _H), _rows(blk, _H)],
        out_shape=[jax.ShapeDtypeStruct((r, _H), jnp.float32),
                   jax.ShapeDtypeStruct((r, _H), jnp.int32)],
    )(x, w1, b1.reshape(1, _H), w2, b2.reshape(1, _H), g.reshape(1, _H),
      bb.reshape(1, _H))


def _edge_step1(g1, e, enc_p, mlp_p, blk, ne_rows):
    noff = e // blk
    ew1, eb1, ew2, eb2, eg, ebb = enc_p
    ws, wr, we, b1, w2, b2, g, bb = mlp_p
    out_shape = [jax.ShapeDtypeStruct((ne_rows, _H), jnp.float32),
                 jax.ShapeDtypeStruct((e, _H), jnp.float32)]
    return pl.pallas_call(
        _edge_step1_body,
        grid=(e // blk,),
        in_specs=[_rows(blk, _H), _rows(blk, _H, off=noff)]
        + [_full((_H, _H)), _full((1, _H))] * 2 + [_full((1, _H))] * 2
        + [_full((_H, _H))] * 3 + [_full((1, _H)), _full((_H, _H)),
                                   _full((1, _H)), _full((1, _H)),
                                   _full((1, _H))],
        out_specs=[_rows(blk, _H), _rows(blk, _H)],
        out_shape=out_shape,
    )(g1, g1, ew1, eb1.reshape(1, _H), ew2, eb2.reshape(1, _H),
      eg.reshape(1, _H), ebb.reshape(1, _H), ws, wr, we,
      b1.reshape(1, _H), w2, b2.reshape(1, _H), g.reshape(1, _H),
      bb.reshape(1, _H))


def _edge_mlp(gath, el, ws, wr, we, b1, w2, b2, g, bb, blk, with_res,
              ne_rows):
    e = el.shape[0]
    noff = e // blk
    out_shape = [jax.ShapeDtypeStruct((ne_rows, _H), jnp.float32)]
    out_specs = [_rows(blk, _H)]
    if with_res:
        out_shape.append(jax.ShapeDtypeStruct((e, _H), jnp.float32))
        out_specs.append(_rows(blk, _H))
    res = pl.pallas_call(
        functools.partial(_edge_mlp_body, with_res),
        grid=(e // blk,),
        in_specs=[_rows(blk, _H), _rows(blk, _H, off=noff), _rows(blk, _H),
                  _full((_H, _H)), _full((_H, _H)), _full((_H, _H)),
                  _full((1, _H)), _full((_H, _H)), _full((1, _H)),
                  _full((1, _H)), _full((1, _H))],
        out_specs=out_specs,
        out_shape=out_shape,
    )(gath, gath, el, ws, wr, we, b1.reshape(1, _H), w2, b2.reshape(1, _H),
      g.reshape(1, _H), bb.reshape(1, _H))
    return res if with_res else res[0]


def _a_spec(blk, c):
    return pl.BlockSpec((1, blk, _H), lambda i, c=c: (c, i, 0))


def _node_mlp(x, acc_a, acc_b, wa, wb, b1, w2, b2, g, bb, blk):
    r = x.shape[0]
    return pl.pallas_call(
        _node_mlp_body,
        grid=(r // blk,),
        in_specs=[_rows(blk, _H), _a_spec(blk, 0), _a_spec(blk, 1),
                  _a_spec(blk, 0), _a_spec(blk, 1), _full((_H, _H)),
                  _full((_H, _H)), _full((1, _H)), _full((_H, _H)),
                  _full((1, _H)), _full((1, _H)), _full((1, _H))],
        out_specs=_rows(blk, _H),
        out_shape=jax.ShapeDtypeStruct((r, _H), jnp.float32),
    )(x, acc_a, acc_a, acc_b, acc_b, wa, wb, b1.reshape(1, _H), w2,
      b2.reshape(1, _H), g.reshape(1, _H), bb.reshape(1, _H))


def _node_dec(x, acc_a, acc_b, wa, wb, b1, w2, b2, g, bb, dw1, db1, dw2,
              db2, blk):
    r = x.shape[0]
    dout = dw2.shape[1]
    return pl.pallas_call(
        _node_dec_body,
        grid=(r // blk,),
        in_specs=[_rows(blk, _H), _a_spec(blk, 0), _a_spec(blk, 1),
                  _a_spec(blk, 0), _a_spec(blk, 1), _full((_H, _H)),
                  _full((_H, _H)), _full((1, _H)), _full((_H, _H)),
                  _full((1, _H)), _full((1, _H)), _full((1, _H)),
                  _full((_H, _H)), _full((1, _H)), _full((_H, dout)),
                  _full((1, dout))],
        out_specs=_rows(blk, dout),
        out_shape=jax.ShapeDtypeStruct((r, dout), jnp.float32),
    )(x, acc_a, acc_a, acc_b, acc_b, wa, wb, b1.reshape(1, _H), w2,
      b2.reshape(1, _H), g.reshape(1, _H), bb.reshape(1, _H), dw1,
      db1.reshape(1, _H), dw2, db2.reshape(1, dout))


def kernel(pos, node, connections, output, mask, noise, mode, params):
    p = params
    pos0 = pos[0]
    node0 = node[0]
    conn = connections[0]
    n = pos0.shape[0]
    e = conn.shape[0]
    half = e // 2
    senders = conn[:, 0]
    receivers = conn[:, 1]
    idx2h = [jnp.concatenate([senders[:half], receivers[:half]]),
             jnp.concatenate([senders[half:], receivers[half:]])]

    nm, ns = p["node_mean"], p["node_std"]
    enc_n = p["enc_node"]
    w1n = enc_n["W"][0] / ns[:, None]
    b1n = enc_n["b"][0] - (nm / ns) @ enc_n["W"][0]
    em, es = p["edge_mean"], p["edge_std"]
    enc_e = p["enc_edge"]
    w1e4 = enc_e["W"][0] / es[:, None]
    b1e = enc_e["b"][0] - (em / es) @ enc_e["W"][0]
    w1e128 = jnp.zeros((_H, _H), jnp.float32).at[:4, :].set(w1e4)

    eblk = 2000
    nblk = 2000

    x_node = jnp.concatenate([node0, pos0], axis=-1)
    feat, table = _enc_node(x_node, w1n, b1n, enc_n["W"][1], enc_n["b"][1],
                            enc_n["ln_g"], enc_n["ln_b"], nblk)

    n_steps = len(p["blocks"])
    edge_lat = [None, None]
    ch_s = 128
    n_ch_s = -(-(half // _NW) // ch_s) | 1
    b_pad = _NW * ch_s * n_ch_s
    recv_pad = [jnp.pad(receivers[:half], (0, b_pad - half),
                        constant_values=n),
                jnp.pad(receivers[half:], (0, b_pad - half),
                        constant_values=n)]
    out = None
    enc_p = (w1e128, b1e, enc_e["W"][1], enc_e["b"][1],
             enc_e["ln_g"], enc_e["ln_b"])
    for t, bp in enumerate(p["blocks"]):
        tab = table if t == 0 else feat
        gath = [_sc_gather(tab, idx2h[0], ch=128),
                _sc_gather(tab, idx2h[1], ch=128)]
        w1 = bp["edge"]["W"][0]
        eres = t < n_steps - 1
        em_p = (w1[:_H], w1[_H:2 * _H], w1[2 * _H:], bp["edge"]["b"][0],
                bp["edge"]["W"][1], bp["edge"]["b"][1], bp["edge"]["ln_g"],
                bp["edge"]["ln_b"])
        nes = [None, None]
        for h in (0, 1):
            if t == 0:
                nes[h], edge_lat[h] = _edge_step1(gath[h], half, enc_p,
                                                  em_p, eblk, b_pad)
            else:
                r = _edge_mlp(gath[h], edge_lat[h], *em_p, eblk,
                              with_res=eres, ne_rows=b_pad)
                if eres:
                    nes[h], edge_lat[h] = r
                else:
                    nes[h] = r
        acc = [_sc_scatter_add(nes[0], recv_pad[0], n, ch=ch_s),
               _sc_scatter_add(nes[1], recv_pad[1], n, ch=ch_s)]
        wn = bp["node"]["W"][0]
        np_ = (wn[:_H], wn[_H:], bp["node"]["b"][0], bp["node"]["W"][1],
               bp["node"]["b"][1], bp["node"]["ln_g"], bp["node"]["ln_b"])
        if t < n_steps - 1:
            feat = _node_mlp(feat, acc[0], acc[1], *np_, nblk)
        else:
            out = _node_dec(feat, acc[0], acc[1], *np_, p["dec"]["W"][0],
                            p["dec"]["b"][0], p["dec"]["W"][1],
                            p["dec"]["b"][1], nblk)
    return out[None]


# speedup vs baseline: 1303.0434x; 1.0136x over previous
import functools

import jax
import jax.numpy as jnp
from jax import lax
from jax.experimental import pallas as pl
from jax.experimental.pallas import tpu as pltpu
from jax.experimental.pallas import tpu_sc as plsc

_NC = 2
_NS = 16
_NW = _NC * _NS
_H = 128


def _sc_gather(table, idx, ch):
    V, D = table.shape
    B = idx.shape[0]
    per_w = B // _NW
    n_ch = -(-per_w // ch)
    if n_ch % 2 == 0:
        n_ch += 1
    assert B % _NW == 0 and per_w % 8 == 0 and ch % 8 == 0 and per_w >= ch
    assert n_ch >= 3 and V % 8 == 0
    rpt = (V // _NS) // 8 * 8
    vlast = V - (_NS - 1) * rpt
    mesh = plsc.VectorSubcoreMesh(core_axis_name="c", subcore_axis_name="s")

    @functools.partial(
        pl.kernel,
        mesh=mesh,
        out_type=jax.ShapeDtypeStruct((B, D), table.dtype),
        scratch_types=[
            pltpu.VMEM((ch,), jnp.int32),
            pltpu.VMEM((ch,), jnp.int32),
            pltpu.VMEM((ch, D), table.dtype),
            pltpu.VMEM((ch, D), table.dtype),
            pltpu.VMEM_SHARED((V, D), table.dtype),
            pltpu.SemaphoreType.DMA,
            pltpu.SemaphoreType.DMA,
        ],
    )
    def k(table_hbm, idx_hbm, out_hbm, ib0, ib1, rb0, rb1, tab, sm0, sm1):
        sid = lax.axis_index("s")
        wid = sid * _NC + lax.axis_index("c")
        base = wid * per_w

        @pl.when(sid < _NS - 1)
        def _():
            pltpu.sync_copy(table_hbm.at[pl.ds(sid * rpt, rpt)],
                            tab.at[pl.ds(sid * rpt, rpt)])

        @pl.when(sid == _NS - 1)
        def _():
            pltpu.sync_copy(table_hbm.at[pl.ds((_NS - 1) * rpt, vlast)],
                            tab.at[pl.ds((_NS - 1) * rpt, vlast)])

        plsc.subcore_barrier()

        def offs(j):
            return base + jnp.minimum(j * ch, per_w - ch)

        def start(j, ib, rb, sm):
            pltpu.sync_copy(idx_hbm.at[pl.ds(offs(j), ch)], ib)
            return pltpu.async_copy(tab.at[ib], rb, sm)

        def drain0():
            pltpu.make_async_copy(table_hbm.at[pl.ds(0, ch)], rb0, sm0).wait()

        start(0, ib0, rb0, sm0)

        def body(p, carry):
            j = 2 * p
            h1 = start(j + 1, ib1, rb1, sm1)
            drain0()
            pltpu.sync_copy(rb0, out_hbm.at[pl.ds(offs(j), ch)])
            start(j + 2, ib0, rb0, sm0)
            h1.wait()
            pltpu.sync_copy(rb1, out_hbm.at[pl.ds(offs(j + 1), ch)])
            return carry

        lax.fori_loop(0, (n_ch - 1) // 2, body, 0)
        drain0()
        pltpu.sync_copy(rb0, out_hbm.at[pl.ds(offs(n_ch - 1), ch)])

    return k(table, idx)


def _sc_scatter_add(rows, idx, n_seg, ch):
    B, D = rows.shape
    per_w = B // _NW
    n_ch = per_w // ch
    assert per_w % ch == 0 and n_seg % 8 == 0
    assert n_ch % 2 == 1 and n_ch >= 3
    n_acc = n_seg + 8
    rpt = (n_acc // _NS) // 8 * 8
    last = n_acc - (_NS - 1) * rpt
    mesh = plsc.VectorSubcoreMesh(core_axis_name="c", subcore_axis_name="s")

    @functools.partial(
        pl.kernel,
        mesh=mesh,
        out_type=jax.ShapeDtypeStruct((_NC, n_acc, D), jnp.float32),
        scratch_types=[
            pltpu.VMEM((ch,), jnp.int32),
            pltpu.VMEM((ch,), jnp.int32),
            pltpu.VMEM((ch, D), jnp.float32),
            pltpu.VMEM((ch, D), jnp.float32),
            pltpu.VMEM_SHARED((n_acc, D), jnp.float32),
            pltpu.SemaphoreType.DMA,
            pltpu.SemaphoreType.DMA,
        ],
    )
    def k(rows_hbm, idx_hbm, out_hbm, ib0, ib1, rb0, rb1, acc, sm0, sm1):
        cid = lax.axis_index("c")
        sid = lax.axis_index("s")
        wid = sid * _NC + cid
        my0 = sid * rpt

        zeros16 = jnp.zeros((16,), jnp.float32)
        lanes_per_row = D // 16

        def zrow(i, carry):
            rb0[i // lanes_per_row,
                pl.ds((i % lanes_per_row) * 16, 16)] = zeros16
            return carry

        lax.fori_loop(0, ch * lanes_per_row, zrow, 0)

        def zfill(row0, nrows):
            full, rem = divmod(nrows, ch)
            for q in range(full):
                pltpu.sync_copy(rb0, acc.at[pl.ds(row0 + q * ch, ch)])
            if rem:
                pltpu.sync_copy(rb0.at[pl.ds(0, rem)],
                                acc.at[pl.ds(row0 + full * ch, rem)])

        @pl.when(sid < _NS - 1)
        def _():
            zfill(my0, rpt)

        @pl.when(sid == _NS - 1)
        def _():
            zfill((_NS - 1) * rpt, last)

        plsc.subcore_barrier()
        base = wid * per_w

        def start(j, ib, rb, sm):
            off = base + j * ch
            pltpu.sync_copy(idx_hbm.at[pl.ds(off, ch)], ib)
            return pltpu.async_copy(rows_hbm.at[pl.ds(off, ch)], rb, sm)

        def drain0():
            pltpu.make_async_copy(rows_hbm.at[pl.ds(0, ch)], rb0, sm0).wait()

        start(0, ib0, rb0, sm0)

        def body(p, carry):
            j = 2 * p
            h1 = start(j + 1, ib1, rb1, sm1)
            drain0()
            pltpu.sync_copy(rb0, acc.at[ib0], add=True)
            start(j + 2, ib0, rb0, sm0)
            h1.wait()
            pltpu.sync_copy(rb1, acc.at[ib1], add=True)
            return carry

        lax.fori_loop(0, (n_ch - 1) // 2, body, 0)
        drain0()
        pltpu.sync_copy(rb0, acc.at[ib0], add=True)
        plsc.subcore_barrier()

        @pl.when(sid < _NS - 1)
        def _():
            pltpu.sync_copy(acc.at[pl.ds(my0, rpt)],
                            out_hbm.at[cid, pl.ds(my0, rpt)])

        @pl.when(sid == _NS - 1)
        def _():
            pltpu.sync_copy(acc.at[pl.ds((_NS - 1) * rpt, last)],
                            out_hbm.at[cid, pl.ds((_NS - 1) * rpt, last)])

    return k(rows, idx)


def _ln(y, g, b):
    mu = jnp.mean(y, axis=-1, keepdims=True)
    yc = y - mu
    var = jnp.mean(yc * yc, axis=-1, keepdims=True)
    return yc * lax.rsqrt(var + 1e-5) * g + b


def _dot(a, b):
    return jnp.dot(a, b, preferred_element_type=jnp.float32)


def _pack_bf16_pair(lo, hi):
    def rnd(x):
        b = lax.bitcast_convert_type(x, jnp.int32)
        b = b + 0x7fff + (lax.shift_right_logical(b, 16) & 1)
        return lax.shift_right_logical(b, 16)
    return rnd(lo) | lax.shift_left(rnd(hi), 16)


def _unpack_bf16_pair(w):
    lo = lax.bitcast_convert_type(lax.shift_left(w, 16), jnp.float32)
    hi = lax.bitcast_convert_type(
        lax.shift_left(lax.shift_right_logical(w, 16), 16), jnp.float32)
    return lo, hi


def _enc_node_body(x_ref, w1_ref, b1_ref, w2_ref, b2_ref, g_ref, bb_ref,
                   o_ref, t_ref):
    x = x_ref[...]
    h = jnp.maximum(_dot(x, w1_ref[...]) + b1_ref[...], 0.0)
    y = _dot(h, w2_ref[...]) + b2_ref[...]
    lat = _ln(y, g_ref[...], bb_ref[...])
    pospad = jnp.concatenate([x[:, -3:], jnp.zeros_like(x[:, 3:])], axis=-1)
    o_ref[...] = lat
    t_ref[...] = _pack_bf16_pair(lat, pospad)


def _edge_step1_body(s_ref, r_ref, ew1_ref, eb1_ref,
                     ew2_ref, eb2_ref, eg_ref, ebb_ref, ws_ref, wr_ref,
                     we_ref, b1_ref, w2_ref, b2_ref, g_ref, bb_ref,
                     ne_ref, eln_ref):
    sf, sp = _unpack_bf16_pair(s_ref[...])
    rf, rp = _unpack_bf16_pair(r_ref[...])
    d = rp - sp
    dist = jnp.sqrt(jnp.sum(d * d, axis=-1, keepdims=True))
    lane = lax.broadcasted_iota(jnp.int32, d.shape, 1)
    x = d + jnp.where(lane == 3, dist, 0.0)
    eh = jnp.maximum(_dot(x, ew1_ref[...]) + eb1_ref[...], 0.0)
    el = _ln(_dot(eh, ew2_ref[...]) + eb2_ref[...], eg_ref[...], ebb_ref[...])
    h = (_dot(sf, ws_ref[...]) + _dot(rf, wr_ref[...])
         + _dot(el, we_ref[...]) + b1_ref[...])
    h = jnp.maximum(h, 0.0)
    y = _dot(h, w2_ref[...]) + b2_ref[...]
    ne = _ln(y, g_ref[...], bb_ref[...])
    ne_ref[...] = ne
    eln_ref[...] = ne + el


def _edge_mlp_body(with_res, *refs):
    if with_res:
        (sf_ref, rf_ref, el_ref, ws_ref, wr_ref, we_ref, b1_ref, w2_ref,
         b2_ref, g_ref, bb_ref, ne_ref, eln_ref) = refs
    else:
        (sf_ref, rf_ref, el_ref, ws_ref, wr_ref, we_ref, b1_ref, w2_ref,
         b2_ref, g_ref, bb_ref, ne_ref) = refs
    el = el_ref[...]
    h = (_dot(sf_ref[...], ws_ref[...]) + _dot(rf_ref[...], wr_ref[...])
         + _dot(el, we_ref[...]) + b1_ref[...])
    h = jnp.maximum(h, 0.0)
    y = _dot(h, w2_ref[...]) + b2_ref[...]
    ne = _ln(y, g_ref[...], bb_ref[...])
    ne_ref[...] = ne
    if with_res:
        eln_ref[...] = ne + el


def _node_feat(x_ref, a0_ref, a1_ref, a2_ref, a3_ref, wa_ref, wb_ref,
               b1_ref, w2_ref, b2_ref, g_ref, bb_ref):
    x = x_ref[...]
    acc = (a0_ref[0] + a1_ref[0]) + (a2_ref[0] + a3_ref[0])
    h = jnp.maximum(_dot(x, wa_ref[...]) + _dot(acc, wb_ref[...])
                    + b1_ref[...], 0.0)
    y = _dot(h, w2_ref[...]) + b2_ref[...]
    return _ln(y, g_ref[...], bb_ref[...]) + x


def _node_mlp_body(x_ref, a0_ref, a1_ref, a2_ref, a3_ref, wa_ref, wb_ref,
                   b1_ref, w2_ref, b2_ref, g_ref, bb_ref, o_ref):
    o_ref[...] = _node_feat(x_ref, a0_ref, a1_ref, a2_ref, a3_ref, wa_ref,
                            wb_ref, b1_ref, w2_ref, b2_ref, g_ref, bb_ref)


def _node_dec_body(x_ref, a0_ref, a1_ref, a2_ref, a3_ref, wa_ref, wb_ref,
                   b1_ref, w2_ref, b2_ref, g_ref, bb_ref, dw1_ref, db1_ref,
                   dw2_ref, db2_ref, o_ref):
    f = _node_feat(x_ref, a0_ref, a1_ref, a2_ref, a3_ref, wa_ref, wb_ref,
                   b1_ref, w2_ref, b2_ref, g_ref, bb_ref)
    h = jnp.maximum(_dot(f, dw1_ref[...]) + db1_ref[...], 0.0)
    o_ref[...] = _dot(h, dw2_ref[...]) + db2_ref[...]


def _full(shape):
    return pl.BlockSpec(shape, lambda i: tuple(0 for _ in shape))


def _rows(blk, d, off=0):
    return pl.BlockSpec((blk, d), lambda i, off=off: (i + off, 0))


def _enc_node(x, w1, b1, w2, b2, g, bb, blk):
    r, din = x.shape
    return pl.pallas_call(
        _enc_node_body,
        grid=(r // blk,),
        in_specs=[_rows(blk, din), _full(w1.shape), _full((1, _H)),
                  _full((_H, _H)), _full((1, _H)), _full((1, _H)),
                  _full((1, _H))],
        out_specs=[_rows(blk, _H), _rows(blk, _H)],
        out_shape=[jax.ShapeDtypeStruct((r, _H), jnp.float32),
                   jax.ShapeDtypeStruct((r, _H), jnp.int32)],
    )(x, w1, b1.reshape(1, _H), w2, b2.reshape(1, _H), g.reshape(1, _H),
      bb.reshape(1, _H))


def _edge_step1(g1, e, enc_p, mlp_p, blk, ne_rows):
    noff = e // blk
    ew1, eb1, ew2, eb2, eg, ebb = enc_p
    ws, wr, we, b1, w2, b2, g, bb = mlp_p
    out_shape = [jax.ShapeDtypeStruct((ne_rows, _H), jnp.float32),
                 jax.ShapeDtypeStruct((e, _H), jnp.float32)]
    return pl.pallas_call(
        _edge_step1_body,
        grid=(e // blk,),
        in_specs=[_rows(blk, _H), _rows(blk, _H, off=noff)]
        + [_full((_H, _H)), _full((1, _H))] * 2 + [_full((1, _H))] * 2
        + [_full((_H, _H))] * 3 + [_full((1, _H)), _full((_H, _H)),
                                   _full((1, _H)), _full((1, _H)),
                                   _full((1, _H))],
        out_specs=[_rows(blk, _H), _rows(blk, _H)],
        out_shape=out_shape,
    )(g1, g1, ew1, eb1.reshape(1, _H), ew2, eb2.reshape(1, _H),
      eg.reshape(1, _H), ebb.reshape(1, _H), ws, wr, we,
      b1.reshape(1, _H), w2, b2.reshape(1, _H), g.reshape(1, _H),
      bb.reshape(1, _H))


def _edge_mlp(gath, el, ws, wr, we, b1, w2, b2, g, bb, blk, with_res,
              ne_rows):
    e = el.shape[0]
    noff = e // blk
    out_shape = [jax.ShapeDtypeStruct((ne_rows, _H), jnp.float32)]
    out_specs = [_rows(blk, _H)]
    if with_res:
        out_shape.append(jax.ShapeDtypeStruct((e, _H), jnp.float32))
        out_specs.append(_rows(blk, _H))
    res = pl.pallas_call(
        functools.partial(_edge_mlp_body, with_res),
        grid=(e // blk,),
        in_specs=[_rows(blk, _H), _rows(blk, _H, off=noff), _rows(blk, _H),
                  _full((_H, _H)), _full((_H, _H)), _full((_H, _H)),
                  _full((1, _H)), _full((_H, _H)), _full((1, _H)),
                  _full((1, _H)), _full((1, _H))],
        out_specs=out_specs,
        out_shape=out_shape,
    )(gath, gath, el, ws, wr, we, b1.reshape(1, _H), w2, b2.reshape(1, _H),
      g.reshape(1, _H), bb.reshape(1, _H))
    return res if with_res else res[0]


def _a_spec(blk, c):
    return pl.BlockSpec((1, blk, _H), lambda i, c=c: (c, i, 0))


def _node_mlp(x, acc_a, acc_b, wa, wb, b1, w2, b2, g, bb, blk):
    r = x.shape[0]
    return pl.pallas_call(
        _node_mlp_body,
        grid=(r // blk,),
        in_specs=[_rows(blk, _H), _a_spec(blk, 0), _a_spec(blk, 1),
                  _a_spec(blk, 0), _a_spec(blk, 1), _full((_H, _H)),
                  _full((_H, _H)), _full((1, _H)), _full((_H, _H)),
                  _full((1, _H)), _full((1, _H)), _full((1, _H))],
        out_specs=_rows(blk, _H),
        out_shape=jax.ShapeDtypeStruct((r, _H), jnp.float32),
    )(x, acc_a, acc_a, acc_b, acc_b, wa, wb, b1.reshape(1, _H), w2,
      b2.reshape(1, _H), g.reshape(1, _H), bb.reshape(1, _H))


def _node_dec(x, acc_a, acc_b, wa, wb, b1, w2, b2, g, bb, dw1, db1, dw2,
              db2, blk):
    r = x.shape[0]
    dout = dw2.shape[1]
    return pl.pallas_call(
        _node_dec_body,
        grid=(r // blk,),
        in_specs=[_rows(blk, _H), _a_spec(blk, 0), _a_spec(blk, 1),
                  _a_spec(blk, 0), _a_spec(blk, 1), _full((_H, _H)),
                  _full((_H, _H)), _full((1, _H)), _full((_H, _H)),
                  _full((1, _H)), _full((1, _H)), _full((1, _H)),
                  _full((_H, _H)), _full((1, _H)), _full((_H, dout)),
                  _full((1, dout))],
        out_specs=_rows(blk, dout),
        out_shape=jax.ShapeDtypeStruct((r, dout), jnp.float32),
    )(x, acc_a, acc_a, acc_b, acc_b, wa, wb, b1.reshape(1, _H), w2,
      b2.reshape(1, _H), g.reshape(1, _H), bb.reshape(1, _H), dw1,
      db1.reshape(1, _H), dw2, db2.reshape(1, dout))


def kernel(pos, node, connections, output, mask, noise, mode, params):
    p = params
    pos0 = pos[0]
    node0 = node[0]
    conn = connections[0]
    n = pos0.shape[0]
    e = conn.shape[0]
    half = e // 2
    senders = conn[:, 0]
    receivers = conn[:, 1]
    idx2h = [jnp.concatenate([senders[:half], receivers[:half]]),
             jnp.concatenate([senders[half:], receivers[half:]])]

    nm, ns = p["node_mean"], p["node_std"]
    enc_n = p["enc_node"]
    w1n = enc_n["W"][0] / ns[:, None]
    b1n = enc_n["b"][0] - (nm / ns) @ enc_n["W"][0]
    em, es = p["edge_mean"], p["edge_std"]
    enc_e = p["enc_edge"]
    w1e4 = enc_e["W"][0] / es[:, None]
    b1e = enc_e["b"][0] - (em / es) @ enc_e["W"][0]
    w1e128 = jnp.zeros((_H, _H), jnp.float32).at[:4, :].set(w1e4)

    eblk = 2000
    nblk = 2000

    x_node = jnp.concatenate([node0, pos0], axis=-1)
    feat, table = _enc_node(x_node, w1n, b1n, enc_n["W"][1], enc_n["b"][1],
                            enc_n["ln_g"], enc_n["ln_b"], nblk)

    n_steps = len(p["blocks"])
    edge_lat = [None, None]
    ch_s = 128
    n_ch_s = -(-(half // _NW) // ch_s) | 1
    b_pad = _NW * ch_s * n_ch_s
    recv_pad = [jnp.pad(receivers[:half], (0, b_pad - half),
                        constant_values=n),
                jnp.pad(receivers[half:], (0, b_pad - half),
                        constant_values=n)]
    out = None
    enc_p = (w1e128, b1e, enc_e["W"][1], enc_e["b"][1],
             enc_e["ln_g"], enc_e["ln_b"])
    for t, bp in enumerate(p["blocks"]):
        tab = table if t == 0 else feat
        gath = [_sc_gather(tab, idx2h[0], ch=128),
                _sc_gather(tab, idx2h[1], ch=128)]
        w1 = bp["edge"]["W"][0]
        eres = t < n_steps - 1
        em_p = (w1[:_H], w1[_H:2 * _H], w1[2 * _H:], bp["edge"]["b"][0],
                bp["edge"]["W"][1], bp["edge"]["b"][1], bp["edge"]["ln_g"],
                bp["edge"]["ln_b"])
        nes = [None, None]
        for h in (0, 1):
            if t == 0:
                nes[h], edge_lat[h] = _edge_step1(gath[h], half, enc_p,
                                                  em_p, eblk, b_pad)
            else:
                r = _edge_mlp(gath[h], edge_lat[h], *em_p, eblk,
                              with_res=eres, ne_rows=b_pad)
                if eres:
                    nes[h], edge_lat[h] = r
                else:
                    nes[h] = r
        acc = [_sc_scatter_add(nes[0], recv_pad[0], n, ch=ch_s),
               _sc_scatter_add(nes[1], recv_pad[1], n, ch=ch_s)]
        wn = bp["node"]["W"][0]
        np_ = (wn[:_H], wn[_H:], bp["node"]["b"][0], bp["node"]["W"][1],
               bp["node"]["b"][1], bp["node"]["ln_g"], bp["node"]["ln_b"])
        if t < n_steps - 1:
            feat = _node_mlp(feat, acc[0], acc[1], *np_, nblk)
        else:
            out = _node_dec(feat, acc[0], acc[1], *np_, p["dec"]["W"][0],
                            p["dec"]["b"][0], p["dec"]["W"][1],
                            p["dec"]["b"][1], nblk)
    return out[None]


# speedup vs baseline: 1371.8996x; 1.0528x over previous
import functools

import jax
import jax.numpy as jnp
from jax import lax
from jax.experimental import pallas as pl
from jax.experimental.pallas import tpu as pltpu
from jax.experimental.pallas import tpu_sc as plsc

_NC = 2
_NS = 16
_NW = _NC * _NS
_H = 128


def _sc_gather(table, idx, ch):
    V, D = table.shape
    B = idx.shape[0]
    per_w = B // _NW
    n_ch = -(-per_w // ch)
    if n_ch % 2 == 0:
        n_ch += 1
    assert B % _NW == 0 and per_w % 8 == 0 and ch % 8 == 0 and per_w >= ch
    assert n_ch >= 3 and V % 8 == 0
    rpt = (V // _NS) // 8 * 8
    vlast = V - (_NS - 1) * rpt
    mesh = plsc.VectorSubcoreMesh(core_axis_name="c", subcore_axis_name="s")

    @functools.partial(
        pl.kernel,
        mesh=mesh,
        out_type=jax.ShapeDtypeStruct((B, D), table.dtype),
        scratch_types=[
            pltpu.VMEM((ch,), jnp.int32),
            pltpu.VMEM((ch,), jnp.int32),
            pltpu.VMEM((ch, D), table.dtype),
            pltpu.VMEM((ch, D), table.dtype),
            pltpu.VMEM_SHARED((V, D), table.dtype),
            pltpu.SemaphoreType.DMA,
            pltpu.SemaphoreType.DMA,
        ],
    )
    def k(table_hbm, idx_hbm, out_hbm, ib0, ib1, rb0, rb1, tab, sm0, sm1):
        sid = lax.axis_index("s")
        wid = sid * _NC + lax.axis_index("c")
        base = wid * per_w

        @pl.when(sid < _NS - 1)
        def _():
            pltpu.sync_copy(table_hbm.at[pl.ds(sid * rpt, rpt)],
                            tab.at[pl.ds(sid * rpt, rpt)])

        @pl.when(sid == _NS - 1)
        def _():
            pltpu.sync_copy(table_hbm.at[pl.ds((_NS - 1) * rpt, vlast)],
                            tab.at[pl.ds((_NS - 1) * rpt, vlast)])

        plsc.subcore_barrier()

        def offs(j):
            return base + jnp.minimum(j * ch, per_w - ch)

        def start(j, ib, rb, sm):
            pltpu.sync_copy(idx_hbm.at[pl.ds(offs(j), ch)], ib)
            return pltpu.async_copy(tab.at[ib], rb, sm)

        def drain0():
            pltpu.make_async_copy(table_hbm.at[pl.ds(0, ch)], rb0, sm0).wait()

        start(0, ib0, rb0, sm0)

        def body(p, carry):
            j = 2 * p
            h1 = start(j + 1, ib1, rb1, sm1)
            drain0()
            pltpu.sync_copy(rb0, out_hbm.at[pl.ds(offs(j), ch)])
            start(j + 2, ib0, rb0, sm0)
            h1.wait()
            pltpu.sync_copy(rb1, out_hbm.at[pl.ds(offs(j + 1), ch)])
            return carry

        lax.fori_loop(0, (n_ch - 1) // 2, body, 0)
        drain0()
        pltpu.sync_copy(rb0, out_hbm.at[pl.ds(offs(n_ch - 1), ch)])

    return k(table, idx)


def _sc_scatter_add(rows, idx, n_seg, ch):
    B, D = rows.shape
    per_w = B // _NW
    n_ch = per_w // ch
    assert per_w % ch == 0 and n_seg % 8 == 0
    assert n_ch % 2 == 1 and n_ch >= 3
    n_acc = n_seg + 8
    rpt = (n_acc // _NS) // 8 * 8
    last = n_acc - (_NS - 1) * rpt
    mesh = plsc.VectorSubcoreMesh(core_axis_name="c", subcore_axis_name="s")

    @functools.partial(
        pl.kernel,
        mesh=mesh,
        out_type=jax.ShapeDtypeStruct((_NC, n_acc, D), jnp.float32),
        scratch_types=[
            pltpu.VMEM((ch,), jnp.int32),
            pltpu.VMEM((ch,), jnp.int32),
            pltpu.VMEM((ch, D), jnp.float32),
            pltpu.VMEM((ch, D), jnp.float32),
            pltpu.VMEM_SHARED((n_acc, D), jnp.float32),
            pltpu.SemaphoreType.DMA,
            pltpu.SemaphoreType.DMA,
        ],
    )
    def k(rows_hbm, idx_hbm, out_hbm, ib0, ib1, rb0, rb1, acc, sm0, sm1):
        cid = lax.axis_index("c")
        sid = lax.axis_index("s")
        wid = sid * _NC + cid
        my0 = sid * rpt

        zeros16 = jnp.zeros((16,), jnp.float32)
        lanes_per_row = D // 16

        def zrow(i, carry):
            rb0[i // lanes_per_row,
                pl.ds((i % lanes_per_row) * 16, 16)] = zeros16
            return carry

        lax.fori_loop(0, ch * lanes_per_row, zrow, 0)

        def zfill(row0, nrows):
            full, rem = divmod(nrows, ch)
            for q in range(full):
                pltpu.sync_copy(rb0, acc.at[pl.ds(row0 + q * ch, ch)])
            if rem:
                pltpu.sync_copy(rb0.at[pl.ds(0, rem)],
                                acc.at[pl.ds(row0 + full * ch, rem)])

        @pl.when(sid < _NS - 1)
        def _():
            zfill(my0, rpt)

        @pl.when(sid == _NS - 1)
        def _():
            zfill((_NS - 1) * rpt, last)

        plsc.subcore_barrier()
        base = wid * per_w

        def start(j, ib, rb, sm):
            off = base + j * ch
            pltpu.sync_copy(idx_hbm.at[pl.ds(off, ch)], ib)
            return pltpu.async_copy(rows_hbm.at[pl.ds(off, ch)], rb, sm)

        def drain0():
            pltpu.make_async_copy(rows_hbm.at[pl.ds(0, ch)], rb0, sm0).wait()

        start(0, ib0, rb0, sm0)

        def body(p, carry):
            j = 2 * p
            h1 = start(j + 1, ib1, rb1, sm1)
            drain0()
            pltpu.sync_copy(rb0, acc.at[ib0], add=True)
            start(j + 2, ib0, rb0, sm0)
            h1.wait()
            pltpu.sync_copy(rb1, acc.at[ib1], add=True)
            return carry

        lax.fori_loop(0, (n_ch - 1) // 2, body, 0)
        drain0()
        pltpu.sync_copy(rb0, acc.at[ib0], add=True)
        plsc.subcore_barrier()

        @pl.when(sid < _NS - 1)
        def _():
            pltpu.sync_copy(acc.at[pl.ds(my0, rpt)],
                            out_hbm.at[cid, pl.ds(my0, rpt)])

        @pl.when(sid == _NS - 1)
        def _():
            pltpu.sync_copy(acc.at[pl.ds((_NS - 1) * rpt, last)],
                            out_hbm.at[cid, pl.ds((_NS - 1) * rpt, last)])

    return k(rows, idx)


def _ln(y, g, b):
    mu = jnp.mean(y, axis=-1, keepdims=True)
    yc = y - mu
    var = jnp.mean(yc * yc, axis=-1, keepdims=True)
    return yc * lax.rsqrt(var + 1e-5) * g + b


def _dot(a, b):
    return jnp.dot(a, b, preferred_element_type=jnp.float32)


def _pack_bf16_pair(lo, hi):
    def rnd(x):
        b = lax.bitcast_convert_type(x, jnp.int32)
        b = b + 0x7fff + (lax.shift_right_logical(b, 16) & 1)
        return lax.shift_right_logical(b, 16)
    return rnd(lo) | lax.shift_left(rnd(hi), 16)


def _unpack_bf16_pair(w):
    lo = lax.bitcast_convert_type(lax.shift_left(w, 16), jnp.float32)
    hi = lax.bitcast_convert_type(
        lax.shift_left(lax.shift_right_logical(w, 16), 16), jnp.float32)
    return lo, hi


def _enc_node_body(x_ref, w1_ref, b1_ref, w2_ref, b2_ref, g_ref, bb_ref,
                   o_ref, t_ref):
    x = x_ref[...]
    h = jnp.maximum(_dot(x, w1_ref[...]) + b1_ref[...], 0.0)
    y = _dot(h, w2_ref[...]) + b2_ref[...]
    lat = _ln(y, g_ref[...], bb_ref[...])
    pospad = jnp.concatenate([x[:, -3:], jnp.zeros_like(x[:, 3:])], axis=-1)
    o_ref[...] = lat
    t_ref[...] = _pack_bf16_pair(lat, pospad)


def _edge_step1_body(s_ref, r_ref, ew1_ref, eb1_ref,
                     ew2_ref, eb2_ref, eg_ref, ebb_ref, ws_ref, wr_ref,
                     we_ref, b1_ref, w2_ref, b2_ref, g_ref, bb_ref,
                     ne_ref, eln_ref):
    sf, sp = _unpack_bf16_pair(s_ref[...])
    rf, rp = _unpack_bf16_pair(r_ref[...])
    d = rp - sp
    dist = jnp.sqrt(jnp.sum(d * d, axis=-1, keepdims=True))
    lane = lax.broadcasted_iota(jnp.int32, d.shape, 1)
    x = d + jnp.where(lane == 3, dist, 0.0)
    eh = jnp.maximum(_dot(x, ew1_ref[...]) + eb1_ref[...], 0.0)
    el = _ln(_dot(eh, ew2_ref[...]) + eb2_ref[...], eg_ref[...], ebb_ref[...])
    h = (_dot(sf, ws_ref[...]) + _dot(rf, wr_ref[...])
         + _dot(el, we_ref[...]) + b1_ref[...])
    h = jnp.maximum(h, 0.0)
    y = _dot(h, w2_ref[...]) + b2_ref[...]
    ne = _ln(y, g_ref[...], bb_ref[...])
    ne_ref[...] = ne
    eln_ref[...] = ne + el


def _edge_mlp_body(with_res, *refs):
    if with_res:
        (sf_ref, rf_ref, el_ref, ws_ref, wr_ref, we_ref, b1_ref, w2_ref,
         b2_ref, g_ref, bb_ref, ne_ref, eln_ref) = refs
    else:
        (sf_ref, rf_ref, el_ref, ws_ref, wr_ref, we_ref, b1_ref, w2_ref,
         b2_ref, g_ref, bb_ref, ne_ref) = refs
    el = el_ref[...]
    h = (_dot(sf_ref[...], ws_ref[...]) + _dot(rf_ref[...], wr_ref[...])
         + _dot(el, we_ref[...]) + b1_ref[...])
    h = jnp.maximum(h, 0.0)
    y = _dot(h, w2_ref[...]) + b2_ref[...]
    ne = _ln(y, g_ref[...], bb_ref[...])
    ne_ref[...] = ne
    if with_res:
        eln_ref[...] = ne + el


def _node_feat(x_ref, a0_ref, a1_ref, a2_ref, a3_ref, wa_ref, wb_ref,
               b1_ref, w2_ref, b2_ref, g_ref, bb_ref):
    x = x_ref[...]
    acc = (a0_ref[0] + a1_ref[0]) + (a2_ref[0] + a3_ref[0])
    h = jnp.maximum(_dot(x, wa_ref[...]) + _dot(acc, wb_ref[...])
                    + b1_ref[...], 0.0)
    y = _dot(h, w2_ref[...]) + b2_ref[...]
    return _ln(y, g_ref[...], bb_ref[...]) + x


def _node_mlp_body(x_ref, a0_ref, a1_ref, a2_ref, a3_ref, wa_ref, wb_ref,
                   b1_ref, w2_ref, b2_ref, g_ref, bb_ref, o_ref):
    o_ref[...] = _node_feat(x_ref, a0_ref, a1_ref, a2_ref, a3_ref, wa_ref,
                            wb_ref, b1_ref, w2_ref, b2_ref, g_ref, bb_ref)


def _node_dec_body(x_ref, a0_ref, a1_ref, a2_ref, a3_ref, wa_ref, wb_ref,
                   b1_ref, w2_ref, b2_ref, g_ref, bb_ref, dw1_ref, db1_ref,
                   dw2_ref, db2_ref, o_ref):
    f = _node_feat(x_ref, a0_ref, a1_ref, a2_ref, a3_ref, wa_ref, wb_ref,
                   b1_ref, w2_ref, b2_ref, g_ref, bb_ref)
    h = jnp.maximum(_dot(f, dw1_ref[...]) + db1_ref[...], 0.0)
    o_ref[...] = _dot(h, dw2_ref[...]) + db2_ref[...]


def _full(shape):
    return pl.BlockSpec(shape, lambda i: tuple(0 for _ in shape))


def _rows(blk, d, off=0):
    return pl.BlockSpec((blk, d), lambda i, off=off: (i + off, 0))


def _enc_node(x, w1, b1, w2, b2, g, bb, blk):
    r, din = x.shape
    return pl.pallas_call(
        _enc_node_body,
        grid=(r // blk,),
        in_specs=[_rows(blk, din), _full(w1.shape), _full((1, _H)),
                  _full((_H, _H)), _full((1, _H)), _full((1, _H)),
                  _full((1, _H))],
        out_specs=[_rows(blk, _H), _rows(blk, _H)],
        out_shape=[jax.ShapeDtypeStruct((r, _H), jnp.float32),
                   jax.ShapeDtypeStruct((r, _H), jnp.int32)],
    )(x, w1, b1.reshape(1, _H), w2, b2.reshape(1, _H), g.reshape(1, _H),
      bb.reshape(1, _H))


def _edge_step1(g1, e, enc_p, mlp_p, blk, ne_rows):
    noff = e // blk
    ew1, eb1, ew2, eb2, eg, ebb = enc_p
    ws, wr, we, b1, w2, b2, g, bb = mlp_p
    out_shape = [jax.ShapeDtypeStruct((ne_rows, _H), jnp.float32),
                 jax.ShapeDtypeStruct((e, _H), jnp.float32)]
    return pl.pallas_call(
        _edge_step1_body,
        grid=(e // blk,),
        in_specs=[_rows(blk, _H), _rows(blk, _H, off=noff)]
        + [_full((_H, _H)), _full((1, _H))] * 2 + [_full((1, _H))] * 2
        + [_full((_H, _H))] * 3 + [_full((1, _H)), _full((_H, _H)),
                                   _full((1, _H)), _full((1, _H)),
                                   _full((1, _H))],
        out_specs=[_rows(blk, _H), _rows(blk, _H)],
        out_shape=out_shape,
    )(g1, g1, ew1, eb1.reshape(1, _H), ew2, eb2.reshape(1, _H),
      eg.reshape(1, _H), ebb.reshape(1, _H), ws, wr, we,
      b1.reshape(1, _H), w2, b2.reshape(1, _H), g.reshape(1, _H),
      bb.reshape(1, _H))


def _edge_mlp(gath, el, ws, wr, we, b1, w2, b2, g, bb, blk, with_res,
              ne_rows):
    e = el.shape[0]
    noff = e // blk
    out_shape = [jax.ShapeDtypeStruct((ne_rows, _H), jnp.float32)]
    out_specs = [_rows(blk, _H)]
    if with_res:
        out_shape.append(jax.ShapeDtypeStruct((e, _H), jnp.float32))
        out_specs.append(_rows(blk, _H))
    res = pl.pallas_call(
        functools.partial(_edge_mlp_body, with_res),
        grid=(e // blk,),
        in_specs=[_rows(blk, _H), _rows(blk, _H, off=noff), _rows(blk, _H),
                  _full((_H, _H)), _full((_H, _H)), _full((_H, _H)),
                  _full((1, _H)), _full((_H, _H)), _full((1, _H)),
                  _full((1, _H)), _full((1, _H))],
        out_specs=out_specs,
        out_shape=out_shape,
    )(gath, gath, el, ws, wr, we, b1.reshape(1, _H), w2, b2.reshape(1, _H),
      g.reshape(1, _H), bb.reshape(1, _H))
    return res if with_res else res[0]


def _a_spec(blk, c):
    return pl.BlockSpec((1, blk, _H), lambda i, c=c: (c, i, 0))


def _node_mlp(x, acc_a, acc_b, wa, wb, b1, w2, b2, g, bb, blk):
    r = x.shape[0]
    return pl.pallas_call(
        _node_mlp_body,
        grid=(r // blk,),
        in_specs=[_rows(blk, _H), _a_spec(blk, 0), _a_spec(blk, 1),
                  _a_spec(blk, 0), _a_spec(blk, 1), _full((_H, _H)),
                  _full((_H, _H)), _full((1, _H)), _full((_H, _H)),
                  _full((1, _H)), _full((1, _H)), _full((1, _H))],
        out_specs=_rows(blk, _H),
        out_shape=jax.ShapeDtypeStruct((r, _H), jnp.float32),
    )(x, acc_a, acc_a, acc_b, acc_b, wa, wb, b1.reshape(1, _H), w2,
      b2.reshape(1, _H), g.reshape(1, _H), bb.reshape(1, _H))


def _node_dec(x, acc_a, acc_b, wa, wb, b1, w2, b2, g, bb, dw1, db1, dw2,
              db2, blk):
    r = x.shape[0]
    dout = dw2.shape[1]
    return pl.pallas_call(
        _node_dec_body,
        grid=(r // blk,),
        in_specs=[_rows(blk, _H), _a_spec(blk, 0), _a_spec(blk, 1),
                  _a_spec(blk, 0), _a_spec(blk, 1), _full((_H, _H)),
                  _full((_H, _H)), _full((1, _H)), _full((_H, _H)),
                  _full((1, _H)), _full((1, _H)), _full((1, _H)),
                  _full((_H, _H)), _full((1, _H)), _full((_H, dout)),
                  _full((1, dout))],
        out_specs=_rows(blk, dout),
        out_shape=jax.ShapeDtypeStruct((r, dout), jnp.float32),
    )(x, acc_a, acc_a, acc_b, acc_b, wa, wb, b1.reshape(1, _H), w2,
      b2.reshape(1, _H), g.reshape(1, _H), bb.reshape(1, _H), dw1,
      db1.reshape(1, _H), dw2, db2.reshape(1, dout))


def kernel(pos, node, connections, output, mask, noise, mode, params):
    p = params
    pos0 = pos[0]
    node0 = node[0]
    conn = connections[0]
    n = pos0.shape[0]
    e = conn.shape[0]
    half = e // 2
    senders = conn[:, 0]
    receivers = conn[:, 1]
    idx2h = [jnp.concatenate([senders[:half], receivers[:half]]),
             jnp.concatenate([senders[half:], receivers[half:]])]

    nm, ns = p["node_mean"], p["node_std"]
    enc_n = p["enc_node"]
    w1n = enc_n["W"][0] / ns[:, None]
    b1n = enc_n["b"][0] - (nm / ns) @ enc_n["W"][0]
    em, es = p["edge_mean"], p["edge_std"]
    enc_e = p["enc_edge"]
    w1e4 = enc_e["W"][0] / es[:, None]
    b1e = enc_e["b"][0] - (em / es) @ enc_e["W"][0]
    w1e128 = jnp.zeros((_H, _H), jnp.float32).at[:4, :].set(w1e4)

    eblk = 4000
    nblk = 2000

    x_node = jnp.concatenate([node0, pos0], axis=-1)
    feat, table = _enc_node(x_node, w1n, b1n, enc_n["W"][1], enc_n["b"][1],
                            enc_n["ln_g"], enc_n["ln_b"], nblk)

    n_steps = len(p["blocks"])
    edge_lat = [None, None]
    ch_s = 128
    n_ch_s = -(-(half // _NW) // ch_s) | 1
    b_pad = _NW * ch_s * n_ch_s
    recv_pad = [jnp.pad(receivers[:half], (0, b_pad - half),
                        constant_values=n),
                jnp.pad(receivers[half:], (0, b_pad - half),
                        constant_values=n)]
    out = None
    enc_p = (w1e128, b1e, enc_e["W"][1], enc_e["b"][1],
             enc_e["ln_g"], enc_e["ln_b"])
    for t, bp in enumerate(p["blocks"]):
        tab = table if t == 0 else feat
        gath = [_sc_gather(tab, idx2h[0], ch=128),
                _sc_gather(tab, idx2h[1], ch=128)]
        w1 = bp["edge"]["W"][0]
        eres = t < n_steps - 1
        em_p = (w1[:_H], w1[_H:2 * _H], w1[2 * _H:], bp["edge"]["b"][0],
                bp["edge"]["W"][1], bp["edge"]["b"][1], bp["edge"]["ln_g"],
                bp["edge"]["ln_b"])
        nes = [None, None]
        for h in (0, 1):
            if t == 0:
                nes[h], edge_lat[h] = _edge_step1(gath[h], half, enc_p,
                                                  em_p, eblk, b_pad)
            else:
                r = _edge_mlp(gath[h], edge_lat[h], *em_p, eblk,
                              with_res=eres, ne_rows=b_pad)
                if eres:
                    nes[h], edge_lat[h] = r
                else:
                    nes[h] = r
        acc = [_sc_scatter_add(nes[0], recv_pad[0], n, ch=ch_s),
               _sc_scatter_add(nes[1], recv_pad[1], n, ch=ch_s)]
        wn = bp["node"]["W"][0]
        np_ = (wn[:_H], wn[_H:], bp["node"]["b"][0], bp["node"]["W"][1],
               bp["node"]["b"][1], bp["node"]["ln_g"], bp["node"]["ln_b"])
        if t < n_steps - 1:
            feat = _node_mlp(feat, acc[0], acc[1], *np_, nblk)
        else:
            out = _node_dec(feat, acc[0], acc[1], *np_, p["dec"]["W"][0],
                            p["dec"]["b"][0], p["dec"]["W"][1],
                            p["dec"]["b"][1], nblk)
    return out[None]


# speedup vs baseline: 1384.3547x; 1.0091x over previous
import functools

import jax
import jax.numpy as jnp
from jax import lax
from jax.experimental import pallas as pl
from jax.experimental.pallas import tpu as pltpu
from jax.experimental.pallas import tpu_sc as plsc

_NC = 2
_NS = 16
_NW = _NC * _NS
_H = 128


def _sc_gather(table, idx, ch):
    V, D = table.shape
    B = idx.shape[0]
    per_w = B // _NW
    n_ch = -(-per_w // ch)
    if n_ch % 2 == 0:
        n_ch += 1
    assert B % _NW == 0 and per_w % 8 == 0 and ch % 8 == 0 and per_w >= ch
    assert n_ch >= 3 and V % 8 == 0
    rpt = (V // _NS) // 8 * 8
    vlast = V - (_NS - 1) * rpt
    mesh = plsc.VectorSubcoreMesh(core_axis_name="c", subcore_axis_name="s")

    @functools.partial(
        pl.kernel,
        mesh=mesh,
        out_type=jax.ShapeDtypeStruct((B, D), table.dtype),
        scratch_types=[
            pltpu.VMEM((ch,), jnp.int32),
            pltpu.VMEM((ch,), jnp.int32),
            pltpu.VMEM((ch, D), table.dtype),
            pltpu.VMEM((ch, D), table.dtype),
            pltpu.VMEM_SHARED((V, D), table.dtype),
            pltpu.SemaphoreType.DMA,
            pltpu.SemaphoreType.DMA,
        ],
    )
    def k(table_hbm, idx_hbm, out_hbm, ib0, ib1, rb0, rb1, tab, sm0, sm1):
        sid = lax.axis_index("s")
        wid = sid * _NC + lax.axis_index("c")
        base = wid * per_w

        @pl.when(sid < _NS - 1)
        def _():
            pltpu.sync_copy(table_hbm.at[pl.ds(sid * rpt, rpt)],
                            tab.at[pl.ds(sid * rpt, rpt)])

        @pl.when(sid == _NS - 1)
        def _():
            pltpu.sync_copy(table_hbm.at[pl.ds((_NS - 1) * rpt, vlast)],
                            tab.at[pl.ds((_NS - 1) * rpt, vlast)])

        plsc.subcore_barrier()

        def offs(j):
            return base + jnp.minimum(j * ch, per_w - ch)

        def start(j, ib, rb, sm):
            pltpu.sync_copy(idx_hbm.at[pl.ds(offs(j), ch)], ib)
            return pltpu.async_copy(tab.at[ib], rb, sm)

        def drain0():
            pltpu.make_async_copy(table_hbm.at[pl.ds(0, ch)], rb0, sm0).wait()

        start(0, ib0, rb0, sm0)

        def body(p, carry):
            j = 2 * p
            h1 = start(j + 1, ib1, rb1, sm1)
            drain0()
            pltpu.sync_copy(rb0, out_hbm.at[pl.ds(offs(j), ch)])
            start(j + 2, ib0, rb0, sm0)
            h1.wait()
            pltpu.sync_copy(rb1, out_hbm.at[pl.ds(offs(j + 1), ch)])
            return carry

        lax.fori_loop(0, (n_ch - 1) // 2, body, 0)
        drain0()
        pltpu.sync_copy(rb0, out_hbm.at[pl.ds(offs(n_ch - 1), ch)])

    return k(table, idx)


def _sc_scatter_add(rows, idx, n_seg, ch):
    B, D = rows.shape
    per_w = B // _NW
    n_ch = per_w // ch
    assert per_w % ch == 0 and n_seg % 8 == 0
    assert n_ch % 2 == 1 and n_ch >= 3
    n_acc = n_seg + 8
    rpt = (n_acc // _NS) // 8 * 8
    last = n_acc - (_NS - 1) * rpt
    mesh = plsc.VectorSubcoreMesh(core_axis_name="c", subcore_axis_name="s")

    @functools.partial(
        pl.kernel,
        mesh=mesh,
        out_type=jax.ShapeDtypeStruct((_NC, n_acc, D), jnp.float32),
        scratch_types=[
            pltpu.VMEM((ch,), jnp.int32),
            pltpu.VMEM((ch,), jnp.int32),
            pltpu.VMEM((ch, D), jnp.float32),
            pltpu.VMEM((ch, D), jnp.float32),
            pltpu.VMEM_SHARED((n_acc, D), jnp.float32),
            pltpu.SemaphoreType.DMA,
            pltpu.SemaphoreType.DMA,
        ],
    )
    def k(rows_hbm, idx_hbm, out_hbm, ib0, ib1, rb0, rb1, acc, sm0, sm1):
        cid = lax.axis_index("c")
        sid = lax.axis_index("s")
        wid = sid * _NC + cid
        my0 = sid * rpt

        zeros16 = jnp.zeros((16,), jnp.float32)
        lanes_per_row = D // 16

        def zrow(i, carry):
            rb0[i // lanes_per_row,
                pl.ds((i % lanes_per_row) * 16, 16)] = zeros16
            return carry

        lax.fori_loop(0, ch * lanes_per_row, zrow, 0)

        def zfill(row0, nrows):
            full, rem = divmod(nrows, ch)
            for q in range(full):
                pltpu.sync_copy(rb0, acc.at[pl.ds(row0 + q * ch, ch)])
            if rem:
                pltpu.sync_copy(rb0.at[pl.ds(0, rem)],
                                acc.at[pl.ds(row0 + full * ch, rem)])

        @pl.when(sid < _NS - 1)
        def _():
            zfill(my0, rpt)

        @pl.when(sid == _NS - 1)
        def _():
            zfill((_NS - 1) * rpt, last)

        plsc.subcore_barrier()
        base = wid * per_w

        def start(j, ib, rb, sm):
            off = base + j * ch
            pltpu.sync_copy(idx_hbm.at[pl.ds(off, ch)], ib)
            return pltpu.async_copy(rows_hbm.at[pl.ds(off, ch)], rb, sm)

        def drain0():
            pltpu.make_async_copy(rows_hbm.at[pl.ds(0, ch)], rb0, sm0).wait()

        start(0, ib0, rb0, sm0)

        def body(p, carry):
            j = 2 * p
            h1 = start(j + 1, ib1, rb1, sm1)
            drain0()
            pltpu.sync_copy(rb0, acc.at[ib0], add=True)
            start(j + 2, ib0, rb0, sm0)
            h1.wait()
            pltpu.sync_copy(rb1, acc.at[ib1], add=True)
            return carry

        lax.fori_loop(0, (n_ch - 1) // 2, body, 0)
        drain0()
        pltpu.sync_copy(rb0, acc.at[ib0], add=True)
        plsc.subcore_barrier()

        @pl.when(sid < _NS - 1)
        def _():
            pltpu.sync_copy(acc.at[pl.ds(my0, rpt)],
                            out_hbm.at[cid, pl.ds(my0, rpt)])

        @pl.when(sid == _NS - 1)
        def _():
            pltpu.sync_copy(acc.at[pl.ds((_NS - 1) * rpt, last)],
                            out_hbm.at[cid, pl.ds((_NS - 1) * rpt, last)])

    return k(rows, idx)


def _ln(y, g, b):
    mu = jnp.mean(y, axis=-1, keepdims=True)
    yc = y - mu
    var = jnp.mean(yc * yc, axis=-1, keepdims=True)
    return yc * lax.rsqrt(var + 1e-5) * g + b


def _dot(a, b):
    return jnp.dot(a, b, preferred_element_type=jnp.float32)


def _pack_bf16_pair(lo, hi):
    def rnd(x):
        b = lax.bitcast_convert_type(x, jnp.int32)
        b = b + 0x7fff + (lax.shift_right_logical(b, 16) & 1)
        return lax.shift_right_logical(b, 16)
    return rnd(lo) | lax.shift_left(rnd(hi), 16)


def _unpack_bf16_pair(w):
    lo = lax.bitcast_convert_type(lax.shift_left(w, 16), jnp.float32)
    hi = lax.bitcast_convert_type(
        lax.shift_left(lax.shift_right_logical(w, 16), 16), jnp.float32)
    return lo, hi


def _enc_node_body(x_ref, w1_ref, b1_ref, w2_ref, b2_ref, g_ref, bb_ref,
                   o_ref, t_ref):
    x = x_ref[...]
    h = jnp.maximum(_dot(x, w1_ref[...]) + b1_ref[...], 0.0)
    y = _dot(h, w2_ref[...]) + b2_ref[...]
    lat = _ln(y, g_ref[...], bb_ref[...])
    pospad = jnp.concatenate([x[:, -3:], jnp.zeros_like(x[:, 3:])], axis=-1)
    o_ref[...] = lat
    t_ref[...] = _pack_bf16_pair(lat, pospad)


def _edge_step1_body(s_ref, r_ref, ew1_ref, eb1_ref,
                     ew2_ref, eb2_ref, eg_ref, ebb_ref, ws_ref, wr_ref,
                     we_ref, b1_ref, w2_ref, b2_ref, g_ref, bb_ref,
                     ne_ref, eln_ref):
    sf, sp = _unpack_bf16_pair(s_ref[...])
    rf, rp = _unpack_bf16_pair(r_ref[...])
    d = rp - sp
    dist = jnp.sqrt(jnp.sum(d * d, axis=-1, keepdims=True))
    lane = lax.broadcasted_iota(jnp.int32, d.shape, 1)
    x = d + jnp.where(lane == 3, dist, 0.0)
    eh = jnp.maximum(_dot(x, ew1_ref[...]) + eb1_ref[...], 0.0)
    el = _ln(_dot(eh, ew2_ref[...]) + eb2_ref[...], eg_ref[...], ebb_ref[...])
    h = (_dot(sf, ws_ref[...]) + _dot(rf, wr_ref[...])
         + _dot(el, we_ref[...]) + b1_ref[...])
    h = jnp.maximum(h, 0.0)
    y = _dot(h, w2_ref[...]) + b2_ref[...]
    ne = _ln(y, g_ref[...], bb_ref[...])
    ne_ref[...] = ne
    eln_ref[...] = ne + el


def _edge_mlp_body(with_res, *refs):
    if with_res:
        (sf_ref, rf_ref, el_ref, ws_ref, wr_ref, we_ref, b1_ref, w2_ref,
         b2_ref, g_ref, bb_ref, ne_ref, eln_ref) = refs
    else:
        (sf_ref, rf_ref, el_ref, ws_ref, wr_ref, we_ref, b1_ref, w2_ref,
         b2_ref, g_ref, bb_ref, ne_ref) = refs
    el = el_ref[...]
    h = (_dot(sf_ref[...], ws_ref[...]) + _dot(rf_ref[...], wr_ref[...])
         + _dot(el, we_ref[...]) + b1_ref[...])
    h = jnp.maximum(h, 0.0)
    y = _dot(h, w2_ref[...]) + b2_ref[...]
    ne = _ln(y, g_ref[...], bb_ref[...])
    ne_ref[...] = ne
    if with_res:
        eln_ref[...] = ne + el


def _node_feat(x_ref, a0_ref, a1_ref, a2_ref, a3_ref, wa_ref, wb_ref,
               b1_ref, w2_ref, b2_ref, g_ref, bb_ref):
    x = x_ref[...]
    acc = (a0_ref[0] + a1_ref[0]) + (a2_ref[0] + a3_ref[0])
    h = jnp.maximum(_dot(x, wa_ref[...]) + _dot(acc, wb_ref[...])
                    + b1_ref[...], 0.0)
    y = _dot(h, w2_ref[...]) + b2_ref[...]
    return _ln(y, g_ref[...], bb_ref[...]) + x


def _node_mlp_body(x_ref, a0_ref, a1_ref, a2_ref, a3_ref, wa_ref, wb_ref,
                   b1_ref, w2_ref, b2_ref, g_ref, bb_ref, o_ref):
    o_ref[...] = _node_feat(x_ref, a0_ref, a1_ref, a2_ref, a3_ref, wa_ref,
                            wb_ref, b1_ref, w2_ref, b2_ref, g_ref, bb_ref)


def _node_dec_body(x_ref, a0_ref, a1_ref, a2_ref, a3_ref, wa_ref, wb_ref,
                   b1_ref, w2_ref, b2_ref, g_ref, bb_ref, dw1_ref, db1_ref,
                   dw2_ref, db2_ref, o_ref):
    f = _node_feat(x_ref, a0_ref, a1_ref, a2_ref, a3_ref, wa_ref, wb_ref,
                   b1_ref, w2_ref, b2_ref, g_ref, bb_ref)
    h = jnp.maximum(_dot(f, dw1_ref[...]) + db1_ref[...], 0.0)
    o_ref[...] = _dot(h, dw2_ref[...]) + db2_ref[...]


def _full(shape):
    return pl.BlockSpec(shape, lambda i: tuple(0 for _ in shape))


def _rows(blk, d, off=0):
    return pl.BlockSpec((blk, d), lambda i, off=off: (i + off, 0))


def _enc_node(x, w1, b1, w2, b2, g, bb, blk):
    r, din = x.shape
    return pl.pallas_call(
        _enc_node_body,
        grid=(r // blk,),
        in_specs=[_rows(blk, din), _full(w1.shape), _full((1, _H)),
                  _full((_H, _H)), _full((1, _H)), _full((1, _H)),
                  _full((1, _H))],
        out_specs=[_rows(blk, _H), _rows(blk, _H)],
        out_shape=[jax.ShapeDtypeStruct((r, _H), jnp.float32),
                   jax.ShapeDtypeStruct((r, _H), jnp.int32)],
    )(x, w1, b1.reshape(1, _H), w2, b2.reshape(1, _H), g.reshape(1, _H),
      bb.reshape(1, _H))


def _edge_step1(g1, e, enc_p, mlp_p, blk, ne_rows):
    noff = e // blk
    ew1, eb1, ew2, eb2, eg, ebb = enc_p
    ws, wr, we, b1, w2, b2, g, bb = mlp_p
    out_shape = [jax.ShapeDtypeStruct((ne_rows, _H), jnp.float32),
                 jax.ShapeDtypeStruct((e, _H), jnp.float32)]
    return pl.pallas_call(
        _edge_step1_body,
        grid=(e // blk,),
        in_specs=[_rows(blk, _H), _rows(blk, _H, off=noff)]
        + [_full((_H, _H)), _full((1, _H))] * 2 + [_full((1, _H))] * 2
        + [_full((_H, _H))] * 3 + [_full((1, _H)), _full((_H, _H)),
                                   _full((1, _H)), _full((1, _H)),
                                   _full((1, _H))],
        out_specs=[_rows(blk, _H), _rows(blk, _H)],
        out_shape=out_shape,
    )(g1, g1, ew1, eb1.reshape(1, _H), ew2, eb2.reshape(1, _H),
      eg.reshape(1, _H), ebb.reshape(1, _H), ws, wr, we,
      b1.reshape(1, _H), w2, b2.reshape(1, _H), g.reshape(1, _H),
      bb.reshape(1, _H))


def _edge_mlp(gath, el, ws, wr, we, b1, w2, b2, g, bb, blk, with_res,
              ne_rows):
    e = el.shape[0]
    noff = e // blk
    out_shape = [jax.ShapeDtypeStruct((ne_rows, _H), jnp.float32)]
    out_specs = [_rows(blk, _H)]
    if with_res:
        out_shape.append(jax.ShapeDtypeStruct((e, _H), jnp.float32))
        out_specs.append(_rows(blk, _H))
    res = pl.pallas_call(
        functools.partial(_edge_mlp_body, with_res),
        grid=(e // blk,),
        in_specs=[_rows(blk, _H), _rows(blk, _H, off=noff), _rows(blk, _H),
                  _full((_H, _H)), _full((_H, _H)), _full((_H, _H)),
                  _full((1, _H)), _full((_H, _H)), _full((1, _H)),
                  _full((1, _H)), _full((1, _H))],
        out_specs=out_specs,
        out_shape=out_shape,
    )(gath, gath, el, ws, wr, we, b1.reshape(1, _H), w2, b2.reshape(1, _H),
      g.reshape(1, _H), bb.reshape(1, _H))
    return res if with_res else res[0]


def _a_spec(blk, c):
    return pl.BlockSpec((1, blk, _H), lambda i, c=c: (c, i, 0))


def _node_mlp(x, acc_a, acc_b, wa, wb, b1, w2, b2, g, bb, blk):
    r = x.shape[0]
    return pl.pallas_call(
        _node_mlp_body,
        grid=(r // blk,),
        in_specs=[_rows(blk, _H), _a_spec(blk, 0), _a_spec(blk, 1),
                  _a_spec(blk, 0), _a_spec(blk, 1), _full((_H, _H)),
                  _full((_H, _H)), _full((1, _H)), _full((_H, _H)),
                  _full((1, _H)), _full((1, _H)), _full((1, _H))],
        out_specs=_rows(blk, _H),
        out_shape=jax.ShapeDtypeStruct((r, _H), jnp.float32),
    )(x, acc_a, acc_a, acc_b, acc_b, wa, wb, b1.reshape(1, _H), w2,
      b2.reshape(1, _H), g.reshape(1, _H), bb.reshape(1, _H))


def _node_dec(x, acc_a, acc_b, wa, wb, b1, w2, b2, g, bb, dw1, db1, dw2,
              db2, blk):
    r = x.shape[0]
    dout = dw2.shape[1]
    return pl.pallas_call(
        _node_dec_body,
        grid=(r // blk,),
        in_specs=[_rows(blk, _H), _a_spec(blk, 0), _a_spec(blk, 1),
                  _a_spec(blk, 0), _a_spec(blk, 1), _full((_H, _H)),
                  _full((_H, _H)), _full((1, _H)), _full((_H, _H)),
                  _full((1, _H)), _full((1, _H)), _full((1, _H)),
                  _full((_H, _H)), _full((1, _H)), _full((_H, dout)),
                  _full((1, dout))],
        out_specs=_rows(blk, dout),
        out_shape=jax.ShapeDtypeStruct((r, dout), jnp.float32),
    )(x, acc_a, acc_a, acc_b, acc_b, wa, wb, b1.reshape(1, _H), w2,
      b2.reshape(1, _H), g.reshape(1, _H), bb.reshape(1, _H), dw1,
      db1.reshape(1, _H), dw2, db2.reshape(1, dout))


def kernel(pos, node, connections, output, mask, noise, mode, params):
    p = params
    pos0 = pos[0]
    node0 = node[0]
    conn = connections[0]
    n = pos0.shape[0]
    e = conn.shape[0]
    half = e // 2
    senders = conn[:, 0]
    receivers = conn[:, 1]
    idx2h = [jnp.concatenate([senders[:half], receivers[:half]]),
             jnp.concatenate([senders[half:], receivers[half:]])]

    nm, ns = p["node_mean"], p["node_std"]
    enc_n = p["enc_node"]
    w1n = enc_n["W"][0] / ns[:, None]
    b1n = enc_n["b"][0] - (nm / ns) @ enc_n["W"][0]
    em, es = p["edge_mean"], p["edge_std"]
    enc_e = p["enc_edge"]
    w1e4 = enc_e["W"][0] / es[:, None]
    b1e = enc_e["b"][0] - (em / es) @ enc_e["W"][0]
    w1e128 = jnp.zeros((_H, _H), jnp.float32).at[:4, :].set(w1e4)

    eblk = 8000
    nblk = 2000

    x_node = jnp.concatenate([node0, pos0], axis=-1)
    feat, table = _enc_node(x_node, w1n, b1n, enc_n["W"][1], enc_n["b"][1],
                            enc_n["ln_g"], enc_n["ln_b"], nblk)

    n_steps = len(p["blocks"])
    edge_lat = [None, None]
    ch_s = 128
    n_ch_s = -(-(half // _NW) // ch_s) | 1
    b_pad = _NW * ch_s * n_ch_s
    recv_pad = [jnp.pad(receivers[:half], (0, b_pad - half),
                        constant_values=n),
                jnp.pad(receivers[half:], (0, b_pad - half),
                        constant_values=n)]
    out = None
    enc_p = (w1e128, b1e, enc_e["W"][1], enc_e["b"][1],
             enc_e["ln_g"], enc_e["ln_b"])
    for t, bp in enumerate(p["blocks"]):
        tab = table if t == 0 else feat
        gath = [_sc_gather(tab, idx2h[0], ch=128),
                _sc_gather(tab, idx2h[1], ch=128)]
        w1 = bp["edge"]["W"][0]
        eres = t < n_steps - 1
        em_p = (w1[:_H], w1[_H:2 * _H], w1[2 * _H:], bp["edge"]["b"][0],
                bp["edge"]["W"][1], bp["edge"]["b"][1], bp["edge"]["ln_g"],
                bp["edge"]["ln_b"])
        nes = [None, None]
        for h in (0, 1):
            if t == 0:
                nes[h], edge_lat[h] = _edge_step1(gath[h], half, enc_p,
                                                  em_p, eblk, b_pad)
            else:
                r = _edge_mlp(gath[h], edge_lat[h], *em_p, eblk,
                              with_res=eres, ne_rows=b_pad)
                if eres:
                    nes[h], edge_lat[h] = r
                else:
                    nes[h] = r
        acc = [_sc_scatter_add(nes[0], recv_pad[0], n, ch=ch_s),
               _sc_scatter_add(nes[1], recv_pad[1], n, ch=ch_s)]
        wn = bp["node"]["W"][0]
        np_ = (wn[:_H], wn[_H:], bp["node"]["b"][0], bp["node"]["W"][1],
               bp["node"]["b"][1], bp["node"]["ln_g"], bp["node"]["ln_b"])
        if t < n_steps - 1:
            feat = _node_mlp(feat, acc[0], acc[1], *np_, nblk)
        else:
            out = _node_dec(feat, acc[0], acc[1], *np_, p["dec"]["W"][0],
                            p["dec"]["b"][0], p["dec"]["W"][1],
                            p["dec"]["b"][1], nblk)
    return out[None]
